# Initial kernel scaffold; baseline (speedup 1.0000x reference)
import jax
import jax.numpy as jnp
from jax.experimental import pallas as pl


def kernel(x, edge_index, edge_attr, edge_indices, edge_indices_f2c, clusters, batches, positions, lengthscales, params):
    raise NotImplementedError("write your pallas kernel here")


# speedup vs baseline: 2.0104x; 2.0104x over previous
import functools

import jax
import jax.numpy as jnp
from jax import lax
from jax.experimental import pallas as pl
from jax.experimental.pallas import tpu as pltpu
from jax.experimental.pallas import tpu_sc as plsc

NC = 2
NS = 16
NW = NC * NS


def _sc_gather(table, idx):
  n, h = table.shape
  b = idx.shape[0]
  per_w = b // NW
  cb = 80
  n_chunks = per_w // cb
  assert per_w * NW == b and n_chunks * cb == per_w

  mesh = plsc.VectorSubcoreMesh(core_axis_name="c", subcore_axis_name="s")

  @functools.partial(
      pl.kernel,
      mesh=mesh,
      out_type=jax.ShapeDtypeStruct((b, h), jnp.float32),
      scratch_types=[
          pltpu.VMEM((cb,), jnp.int32),
          pltpu.VMEM((cb, h), jnp.float32),
          pltpu.SemaphoreType.DMA,
      ],
  )
  def k(table_hbm, idx_hbm, out_hbm, idx_v, rows_v, sem):
    wid = lax.axis_index("s") * NC + lax.axis_index("c")
    base = wid * per_w

    def body(j, carry):
      off = base + j * cb
      pltpu.sync_copy(idx_hbm.at[pl.ds(off, cb)], idx_v)
      pltpu.async_copy(table_hbm.at[idx_v], rows_v, sem).wait()
      pltpu.sync_copy(rows_v, out_hbm.at[pl.ds(off, cb)])
      return carry

    lax.fori_loop(0, n_chunks, body, 0)

  return k(table, idx)


def _sc_scatter_add(vals, dst, n, h):
  e = dst.shape[0]
  per_w = e // NW
  cb = 80
  n_chunks = per_w // cb
  zr = 640
  spacing = 624
  assert per_w * NW == e and n_chunks * cb == per_w
  assert spacing % 8 == 0 and spacing <= zr and spacing * (NS - 1) + zr == n
  n_zchunks = zr // cb
  use_vals = vals is not None

  mesh = plsc.VectorSubcoreMesh(core_axis_name="c", subcore_axis_name="s")

  @functools.partial(
      pl.kernel,
      mesh=mesh,
      out_type=jax.ShapeDtypeStruct((NC * n, h), jnp.float32),
      scratch_types=[
          pltpu.VMEM((cb,), jnp.int32),
          pltpu.VMEM((cb, h), jnp.float32),
          pltpu.VMEM_SHARED((n, h), jnp.float32),
      ],
  )
  def k(*refs):
    if use_vals:
      vals_hbm, dst_hbm, sums_hbm, idx_v, rows_v, s_sh = refs
    else:
      dst_hbm, sums_hbm, idx_v, rows_v, s_sh = refs
    cid = lax.axis_index("c")
    sid = lax.axis_index("s")
    wid = sid * NC + cid

    fill = jnp.zeros((16,), jnp.float32)
    for r in range(cb):
      for q in range(h // 16):
        rows_v[r, pl.ds(q * 16, 16)] = fill

    base_r = sid * spacing
    for zc in range(n_zchunks):
      pltpu.sync_copy(rows_v, s_sh.at[pl.ds(base_r + zc * cb, cb)])
    plsc.subcore_barrier()

    if not use_vals:
      one = jnp.ones((16,), jnp.float32)
      for r in range(cb):
        for q in range(h // 16):
          rows_v[r, pl.ds(q * 16, 16)] = one

    def body(j, carry):
      off = wid * per_w + j * cb
      pltpu.sync_copy(dst_hbm.at[pl.ds(off, cb)], idx_v)
      if use_vals:
        pltpu.sync_copy(vals_hbm.at[pl.ds(off, cb)], rows_v)
      pltpu.sync_copy(rows_v, s_sh.at[idx_v], add=True)
      return carry

    lax.fori_loop(0, n_chunks, body, 0)
    plsc.subcore_barrier()

    out_base = cid * n + sid * spacing
    for zc in range(n_zchunks):
      pltpu.sync_copy(s_sh.at[pl.ds(base_r + zc * cb, cb)], rows_v)
      pltpu.sync_copy(rows_v, sums_hbm.at[pl.ds(out_base + zc * cb, cb)])

  if use_vals:
    out = k(vals, dst)
  else:
    out = k(dst)
  return out.reshape(NC, n, h)


def _elu(t):
  return jnp.where(t > 0, t, jnp.exp(jnp.minimum(t, 0.0)) - 1.0)


def _ln(r, gam, bet):
  mu = jnp.mean(r, axis=-1, keepdims=True)
  d = r - mu
  var = jnp.mean(d * d, axis=-1, keepdims=True)
  return d * lax.rsqrt(var + 1e-5) * gam + bet


def _edge_body(g0, g1, ea, w0a, w0b, w0c, w1, b0, b1, gam, bet, out):
  eav = ea[...]
  t = (jnp.dot(g0[...], w0a[...], preferred_element_type=jnp.float32)
       + jnp.dot(g1[...], w0b[...], preferred_element_type=jnp.float32)
       + jnp.dot(eav, w0c[...], preferred_element_type=jnp.float32)
       + b0[...])
  t = _elu(t)
  t = jnp.dot(t, w1[...], preferred_element_type=jnp.float32) + b1[...]
  out[...] = _ln(eav + t, gam[...], bet[...])


def _edge_mlp(g, ea, w0a, w0b, w0c, w1, b0, b1, gam, bet):
  e, h = ea.shape
  be = 512
  grid = e // be
  assert grid * be == e
  wspec = pl.BlockSpec((h, h), lambda i: (0, 0))
  vspec = pl.BlockSpec((1, h), lambda i: (0, 0))
  return pl.pallas_call(
      _edge_body,
      grid=(grid,),
      in_specs=[
          pl.BlockSpec((be, h), lambda i: (i, 0)),
          pl.BlockSpec((be, h), lambda i, g_=grid: (i + g_, 0)),
          pl.BlockSpec((be, h), lambda i: (i, 0)),
          wspec, wspec, wspec, wspec, vspec, vspec, vspec, vspec,
      ],
      out_specs=pl.BlockSpec((be, h), lambda i: (i, 0)),
      out_shape=jax.ShapeDtypeStruct((e, h), jnp.float32),
      compiler_params=pltpu.CompilerParams(
          dimension_semantics=("arbitrary",)),
  )(g, g, ea, w0a, w0b, w0c, w1, b0, b1, gam, bet)


def _node_body(final, xb, sums, cnts, wa, wb, w1, b0, b1, gam, bet,
               wo, bo, out, *maybe_proj):
  xv = xb[...]
  s = sums[0] + sums[1]
  c = jnp.maximum(cnts[0] + cnts[1], 1.0)
  agg = s / c
  t = (jnp.dot(xv, wa[...], preferred_element_type=jnp.float32)
       + jnp.dot(agg, wb[...], preferred_element_type=jnp.float32)
       + b0[...])
  t = _elu(t)
  t = jnp.dot(t, w1[...], preferred_element_type=jnp.float32) + b1[...]
  xn = _ln(xv + t, gam[...], bet[...])
  out[...] = xn
  if final:
    proj = jnp.dot(xn, wo[...], preferred_element_type=jnp.float32) + bo[...]
    maybe_proj[0][...] = _elu(proj)


def _node_mlp(x, sums, cnts, wa, wb, w1, b0, b1, gam, bet, wo, bo, final):
  n, h = x.shape
  bn = 512
  grid = pl.cdiv(n, bn)
  wspec = pl.BlockSpec((h, h), lambda i: (0, 0))
  vspec = pl.BlockSpec((1, h), lambda i: (0, 0))
  out_shape = [jax.ShapeDtypeStruct((n, h), jnp.float32)]
  out_specs = [pl.BlockSpec((bn, h), lambda i: (i, 0))]
  if final:
    out_shape.append(jax.ShapeDtypeStruct((n, h), jnp.float32))
    out_specs.append(pl.BlockSpec((bn, h), lambda i: (i, 0)))
  res = pl.pallas_call(
      functools.partial(_node_body, final),
      grid=(grid,),
      in_specs=[
          pl.BlockSpec((bn, h), lambda i: (i, 0)),
          pl.BlockSpec((NC, bn, h), lambda i: (0, i, 0)),
          pl.BlockSpec((NC, bn, h), lambda i: (0, i, 0)),
          wspec, wspec, wspec, vspec, vspec, vspec, vspec,
          wspec, vspec,
      ],
      out_specs=out_specs,
      out_shape=out_shape,
      compiler_params=pltpu.CompilerParams(
          dimension_semantics=("arbitrary",)),
  )(x, sums, cnts, wa, wb, w1, b0, b1, gam, bet, wo, bo)
  return res if final else res[0]


def kernel(x, edge_index, edge_attr, edge_indices, edge_indices_f2c, clusters,
           batches, positions, lengthscales, params):
  n, h = x.shape
  ei = edge_indices[0]
  e = ei.shape[1]
  idx_flat = ei.reshape(-1)
  dst = idx_flat[e:]

  wout_pad = jnp.zeros((h, h), jnp.float32).at[:, :params['wout'].shape[1]].set(
      params['wout'])
  bout_pad = jnp.zeros((1, h), jnp.float32).at[:, :params['bout'].shape[0]].set(
      params['bout'][None, :])

  def row(v):
    return v.reshape(1, h)

  ea = edge_attr
  out_proj = None
  n_mp = 2
  cnts = _sc_scatter_add(None, dst, n, h)
  for i in range(n_mp):
    w0 = params['ew0_%d' % i]
    g = _sc_gather(x, idx_flat)
    ea = _edge_mlp(g, ea, w0[:h], w0[h:2 * h], w0[2 * h:],
                   params['ew1_%d' % i], row(params['eb0_%d' % i]),
                   row(params['eb1_%d' % i]), row(params['eg_%d' % i]),
                   row(params['ebt_%d' % i]))
    sums = _sc_scatter_add(ea, dst, n, h)
    nw0 = params['nw0_%d' % i]
    final = i == n_mp - 1
    res = _node_mlp(x, sums, cnts, nw0[:h], nw0[h:],
                    params['nw1_%d' % i], row(params['nb0_%d' % i]),
                    row(params['nb1_%d' % i]), row(params['ng_%d' % i]),
                    row(params['nbt_%d' % i]), wout_pad, bout_pad, final)
    if final:
      x, out_proj = res
    else:
      x = res

  return (out_proj[:, :params['wout'].shape[1]], ei)


# speedup vs baseline: 2.5896x; 1.2881x over previous
import functools

import jax
import jax.numpy as jnp
from jax import lax
from jax.experimental import pallas as pl
from jax.experimental.pallas import tpu as pltpu
from jax.experimental.pallas import tpu_sc as plsc

NC = 2
NS = 16
NW = NC * NS


def _sc_gather(table, idx):
  n, h = table.shape
  b = idx.shape[0]
  per_w = b // NW
  cb = 80
  n_chunks = per_w // cb
  assert per_w * NW == b and n_chunks * cb == per_w

  mesh = plsc.VectorSubcoreMesh(core_axis_name="c", subcore_axis_name="s")

  assert n_chunks % 2 == 0

  @functools.partial(
      pl.kernel,
      mesh=mesh,
      out_type=jax.ShapeDtypeStruct((b, h), jnp.float32),
      scratch_types=[
          pltpu.VMEM((cb,), jnp.int32),
          pltpu.VMEM((cb,), jnp.int32),
          pltpu.VMEM((cb, h), jnp.float32),
          pltpu.VMEM((cb, h), jnp.float32),
          pltpu.SemaphoreType.DMA,
          pltpu.SemaphoreType.DMA,
          pltpu.SemaphoreType.DMA,
          pltpu.SemaphoreType.DMA,
          pltpu.SemaphoreType.DMA,
          pltpu.SemaphoreType.DMA,
      ],
  )
  def k(table_hbm, idx_hbm, out_hbm, ib0, ib1, rb0, rb1,
        si0, si1, sg0, sg1, so0, so1):
    wid = lax.axis_index("s") * NC + lax.axis_index("c")
    base = wid * per_w
    ibs, rbs = (ib0, ib1), (rb0, rb1)
    sis, sgs, sos = (si0, si1), (sg0, sg1), (so0, so1)

    for bb in range(2):
      pltpu.async_copy(idx_hbm.at[pl.ds(base + bb * cb, cb)], ibs[bb], sis[bb])

    def outer(g, carry):
      for bb in range(2):
        jj = g * 2 + bb
        off = base + jj * cb
        @pl.when(jj >= 2)
        def _():
          pltpu.make_async_copy(
              rbs[bb], out_hbm.at[pl.ds(off - 2 * cb, cb)], sos[bb]).wait()
        pltpu.make_async_copy(
            idx_hbm.at[pl.ds(off, cb)], ibs[bb], sis[bb]).wait()
        pltpu.async_copy(table_hbm.at[ibs[bb]], rbs[bb], sgs[bb]).wait()
        pltpu.async_copy(rbs[bb], out_hbm.at[pl.ds(off, cb)], sos[bb])
        @pl.when(jj + 2 < n_chunks)
        def _():
          pltpu.async_copy(
              idx_hbm.at[pl.ds(off + 2 * cb, cb)], ibs[bb], sis[bb])
      return carry

    lax.fori_loop(0, n_chunks // 2, outer, 0)
    for bb in range(2):
      off = base + (n_chunks - 2 + bb) * cb
      pltpu.make_async_copy(
          rbs[bb], out_hbm.at[pl.ds(off, cb)], sos[bb]).wait()

  return k(table, idx)


def _sc_scatter_add(vals, dst, n, h):
  e = dst.shape[0]
  per_w = e // NW
  cb = 80
  n_chunks = per_w // cb
  zr = 640
  spacing = 624
  assert per_w * NW == e and n_chunks * cb == per_w
  assert spacing % 8 == 0 and spacing <= zr and spacing * (NS - 1) + zr == n
  n_zchunks = zr // cb
  use_vals = vals is not None

  mesh = plsc.VectorSubcoreMesh(core_axis_name="c", subcore_axis_name="s")

  @functools.partial(
      pl.kernel,
      mesh=mesh,
      out_type=jax.ShapeDtypeStruct((NC * n, h), jnp.float32),
      scratch_types=[
          pltpu.VMEM((cb,), jnp.int32),
          pltpu.VMEM((cb,), jnp.int32),
          pltpu.VMEM((cb, h), jnp.float32),
          pltpu.VMEM((cb, h), jnp.float32),
          pltpu.SemaphoreType.DMA,
          pltpu.SemaphoreType.DMA,
          pltpu.SemaphoreType.DMA,
          pltpu.SemaphoreType.DMA,
          pltpu.VMEM_SHARED((n, h), jnp.float32),
      ],
  )
  def k(*refs):
    if use_vals:
      (vals_hbm, dst_hbm, sums_hbm, ib0, ib1, rb0, rb1,
       si0, si1, sr0, sr1, s_sh) = refs
    else:
      (dst_hbm, sums_hbm, ib0, ib1, rb0, rb1,
       si0, si1, sr0, sr1, s_sh) = refs
    cid = lax.axis_index("c")
    sid = lax.axis_index("s")
    wid = sid * NC + cid
    ibs, rbs = (ib0, ib1), (rb0, rb1)
    sis, srs = (si0, si1), (sr0, sr1)

    fill = jnp.zeros((16,), jnp.float32)
    for rb in rbs:
      for r in range(cb):
        for q in range(h // 16):
          rb[r, pl.ds(q * 16, 16)] = fill

    base_r = sid * spacing
    for zc in range(n_zchunks):
      pltpu.sync_copy(rbs[zc % 2], s_sh.at[pl.ds(base_r + zc * cb, cb)])
    plsc.subcore_barrier()

    if not use_vals:
      one = jnp.ones((16,), jnp.float32)
      for rb in rbs:
        for r in range(cb):
          for q in range(h // 16):
            rb[r, pl.ds(q * 16, 16)] = one

    for bb in range(2):
      off = wid * per_w + bb * cb
      pltpu.async_copy(dst_hbm.at[pl.ds(off, cb)], ibs[bb], sis[bb])
      if use_vals:
        pltpu.async_copy(vals_hbm.at[pl.ds(off, cb)], rbs[bb], srs[bb])

    def chunk_work(jj, bb, may_prefetch):
      off = wid * per_w + jj * cb
      pltpu.make_async_copy(
          dst_hbm.at[pl.ds(off, cb)], ibs[bb], sis[bb]).wait()
      if use_vals:
        pltpu.make_async_copy(
            vals_hbm.at[pl.ds(off, cb)], rbs[bb], srs[bb]).wait()
      pltpu.sync_copy(rbs[bb], s_sh.at[ibs[bb]], add=True)
      if may_prefetch:
        @pl.when(jj + 2 < n_chunks)
        def _():
          pltpu.async_copy(
              dst_hbm.at[pl.ds(off + 2 * cb, cb)], ibs[bb], sis[bb])
          if use_vals:
            pltpu.async_copy(
                vals_hbm.at[pl.ds(off + 2 * cb, cb)], rbs[bb], srs[bb])

    def body(g, carry):
      for bb in range(2):
        chunk_work(g * 2 + bb, bb, True)
      return carry

    lax.fori_loop(0, n_chunks // 2, body, 0)
    if n_chunks % 2:
      chunk_work(n_chunks - 1, (n_chunks - 1) % 2, False)
    plsc.subcore_barrier()

    out_base = cid * n + sid * spacing
    for zc in range(n_zchunks):
      pltpu.sync_copy(s_sh.at[pl.ds(base_r + zc * cb, cb)], rbs[zc % 2])
      pltpu.sync_copy(rbs[zc % 2], sums_hbm.at[pl.ds(out_base + zc * cb, cb)])

  if use_vals:
    out = k(vals, dst)
  else:
    out = k(dst)
  return out.reshape(NC, n, h)


def _elu(t):
  return jnp.where(t > 0, t, jnp.exp(jnp.minimum(t, 0.0)) - 1.0)


def _ln(r, gam, bet):
  mu = jnp.mean(r, axis=-1, keepdims=True)
  d = r - mu
  var = jnp.mean(d * d, axis=-1, keepdims=True)
  return d * lax.rsqrt(var + 1e-5) * gam + bet


def _edge_body(g0, g1, ea, w0a, w0b, w0c, w1, b0, b1, gam, bet, out):
  eav = ea[...]
  t = (jnp.dot(g0[...], w0a[...], preferred_element_type=jnp.float32)
       + jnp.dot(g1[...], w0b[...], preferred_element_type=jnp.float32)
       + jnp.dot(eav, w0c[...], preferred_element_type=jnp.float32)
       + b0[...])
  t = _elu(t)
  t = jnp.dot(t, w1[...], preferred_element_type=jnp.float32) + b1[...]
  out[...] = _ln(eav + t, gam[...], bet[...])


def _edge_mlp(g, ea, w0a, w0b, w0c, w1, b0, b1, gam, bet):
  e, h = ea.shape
  be = 512
  grid = e // be
  assert grid * be == e
  wspec = pl.BlockSpec((h, h), lambda i: (0, 0))
  vspec = pl.BlockSpec((1, h), lambda i: (0, 0))
  return pl.pallas_call(
      _edge_body,
      grid=(grid,),
      in_specs=[
          pl.BlockSpec((be, h), lambda i: (i, 0)),
          pl.BlockSpec((be, h), lambda i, g_=grid: (i + g_, 0)),
          pl.BlockSpec((be, h), lambda i: (i, 0)),
          wspec, wspec, wspec, wspec, vspec, vspec, vspec, vspec,
      ],
      out_specs=pl.BlockSpec((be, h), lambda i: (i, 0)),
      out_shape=jax.ShapeDtypeStruct((e, h), jnp.float32),
      compiler_params=pltpu.CompilerParams(
          dimension_semantics=("arbitrary",)),
  )(g, g, ea, w0a, w0b, w0c, w1, b0, b1, gam, bet)


def _node_body(final, xb, sums, cnts, wa, wb, w1, b0, b1, gam, bet,
               wo, bo, out, *maybe_proj):
  xv = xb[...]
  s = sums[0] + sums[1]
  c = jnp.maximum(cnts[0] + cnts[1], 1.0)
  agg = s / c
  t = (jnp.dot(xv, wa[...], preferred_element_type=jnp.float32)
       + jnp.dot(agg, wb[...], preferred_element_type=jnp.float32)
       + b0[...])
  t = _elu(t)
  t = jnp.dot(t, w1[...], preferred_element_type=jnp.float32) + b1[...]
  xn = _ln(xv + t, gam[...], bet[...])
  out[...] = xn
  if final:
    proj = jnp.dot(xn, wo[...], preferred_element_type=jnp.float32) + bo[...]
    maybe_proj[0][...] = _elu(proj)


def _node_mlp(x, sums, cnts, wa, wb, w1, b0, b1, gam, bet, wo, bo, final):
  n, h = x.shape
  bn = 512
  grid = pl.cdiv(n, bn)
  wspec = pl.BlockSpec((h, h), lambda i: (0, 0))
  vspec = pl.BlockSpec((1, h), lambda i: (0, 0))
  out_shape = [jax.ShapeDtypeStruct((n, h), jnp.float32)]
  out_specs = [pl.BlockSpec((bn, h), lambda i: (i, 0))]
  if final:
    out_shape.append(jax.ShapeDtypeStruct((n, h), jnp.float32))
    out_specs.append(pl.BlockSpec((bn, h), lambda i: (i, 0)))
  res = pl.pallas_call(
      functools.partial(_node_body, final),
      grid=(grid,),
      in_specs=[
          pl.BlockSpec((bn, h), lambda i: (i, 0)),
          pl.BlockSpec((NC, bn, h), lambda i: (0, i, 0)),
          pl.BlockSpec((NC, bn, h), lambda i: (0, i, 0)),
          wspec, wspec, wspec, vspec, vspec, vspec, vspec,
          wspec, vspec,
      ],
      out_specs=out_specs,
      out_shape=out_shape,
      compiler_params=pltpu.CompilerParams(
          dimension_semantics=("arbitrary",)),
  )(x, sums, cnts, wa, wb, w1, b0, b1, gam, bet, wo, bo)
  return res if final else res[0]


def kernel(x, edge_index, edge_attr, edge_indices, edge_indices_f2c, clusters,
           batches, positions, lengthscales, params):
  n, h = x.shape
  ei = edge_indices[0]
  e = ei.shape[1]
  idx_flat = ei.reshape(-1)
  dst = idx_flat[e:]

  wout_pad = jnp.zeros((h, h), jnp.float32).at[:, :params['wout'].shape[1]].set(
      params['wout'])
  bout_pad = jnp.zeros((1, h), jnp.float32).at[:, :params['bout'].shape[0]].set(
      params['bout'][None, :])

  def row(v):
    return v.reshape(1, h)

  ea = edge_attr
  out_proj = None
  n_mp = 2
  cnts = _sc_scatter_add(None, dst, n, h)
  for i in range(n_mp):
    w0 = params['ew0_%d' % i]
    g = _sc_gather(x, idx_flat)
    ea = _edge_mlp(g, ea, w0[:h], w0[h:2 * h], w0[2 * h:],
                   params['ew1_%d' % i], row(params['eb0_%d' % i]),
                   row(params['eb1_%d' % i]), row(params['eg_%d' % i]),
                   row(params['ebt_%d' % i]))
    sums = _sc_scatter_add(ea, dst, n, h)
    nw0 = params['nw0_%d' % i]
    final = i == n_mp - 1
    res = _node_mlp(x, sums, cnts, nw0[:h], nw0[h:],
                    params['nw1_%d' % i], row(params['nb0_%d' % i]),
                    row(params['nb1_%d' % i]), row(params['ng_%d' % i]),
                    row(params['nbt_%d' % i]), wout_pad, bout_pad, final)
    if final:
      x, out_proj = res
    else:
      x = res

  return (out_proj[:, :params['wout'].shape[1]], ei)


# speedup vs baseline: 3.1009x; 1.1975x over previous
import functools

import jax
import jax.numpy as jnp
from jax import lax
from jax.experimental import pallas as pl
from jax.experimental.pallas import tpu as pltpu
from jax.experimental.pallas import tpu_sc as plsc

NC = 2
NS = 16
NW = NC * NS


def _sc_gather(table, idx):
  n, h = table.shape
  b = idx.shape[0]
  per_w = b // NW
  cb = next(c for c in (80, 40, 16, 8) if per_w % c == 0)
  n_chunks = per_w // cb
  assert per_w * NW == b and n_chunks * cb == per_w

  mesh = plsc.VectorSubcoreMesh(core_axis_name="c", subcore_axis_name="s")

  assert n_chunks >= 2

  @functools.partial(
      pl.kernel,
      mesh=mesh,
      out_type=jax.ShapeDtypeStruct((b, h), jnp.float32),
      scratch_types=[
          pltpu.VMEM((cb,), jnp.int32),
          pltpu.VMEM((cb,), jnp.int32),
          pltpu.VMEM((cb, h), jnp.float32),
          pltpu.VMEM((cb, h), jnp.float32),
          pltpu.SemaphoreType.DMA,
          pltpu.SemaphoreType.DMA,
          pltpu.SemaphoreType.DMA,
          pltpu.SemaphoreType.DMA,
          pltpu.SemaphoreType.DMA,
          pltpu.SemaphoreType.DMA,
      ],
  )
  def k(table_hbm, idx_hbm, out_hbm, ib0, ib1, rb0, rb1,
        si0, si1, sg0, sg1, so0, so1):
    wid = lax.axis_index("s") * NC + lax.axis_index("c")
    base = wid * per_w
    ibs, rbs = (ib0, ib1), (rb0, rb1)
    sis, sgs, sos = (si0, si1), (sg0, sg1), (so0, so1)

    for bb in range(2):
      pltpu.async_copy(idx_hbm.at[pl.ds(base + bb * cb, cb)], ibs[bb], sis[bb])

    def chunk_work(jj, bb, may_prefetch):
      off = base + jj * cb
      @pl.when(jj >= 2)
      def _():
        pltpu.make_async_copy(
            rbs[bb], out_hbm.at[pl.ds(off - 2 * cb, cb)], sos[bb]).wait()
      pltpu.make_async_copy(
          idx_hbm.at[pl.ds(off, cb)], ibs[bb], sis[bb]).wait()
      pltpu.async_copy(table_hbm.at[ibs[bb]], rbs[bb], sgs[bb]).wait()
      pltpu.async_copy(rbs[bb], out_hbm.at[pl.ds(off, cb)], sos[bb])
      if may_prefetch:
        @pl.when(jj + 2 < n_chunks)
        def _():
          pltpu.async_copy(
              idx_hbm.at[pl.ds(off + 2 * cb, cb)], ibs[bb], sis[bb])

    def outer(g, carry):
      for bb in range(2):
        chunk_work(g * 2 + bb, bb, True)
      return carry

    lax.fori_loop(0, n_chunks // 2, outer, 0)
    if n_chunks % 2:
      chunk_work(n_chunks - 1, (n_chunks - 1) % 2, False)
    for bb in range(2):
      jj_last = n_chunks - 1 - ((n_chunks - 1 - bb) % 2)
      off = base + jj_last * cb
      pltpu.make_async_copy(
          rbs[bb], out_hbm.at[pl.ds(off, cb)], sos[bb]).wait()

  return k(table, idx)


def _sc_scatter_add(vals, dst, n, h):
  e = dst.shape[0]
  per_w = e // NW
  cb = next(c for c in (80, 40, 16, 8) if per_w % c == 0)
  n_chunks = per_w // cb
  zr = 640
  spacing = 624
  assert per_w * NW == e and n_chunks * cb == per_w
  assert spacing % 8 == 0 and spacing <= zr and spacing * (NS - 1) + zr == n
  n_zchunks = zr // cb
  use_vals = vals is not None

  mesh = plsc.VectorSubcoreMesh(core_axis_name="c", subcore_axis_name="s")

  @functools.partial(
      pl.kernel,
      mesh=mesh,
      out_type=jax.ShapeDtypeStruct((NC * n, h), jnp.float32),
      scratch_types=[
          pltpu.VMEM((cb,), jnp.int32),
          pltpu.VMEM((cb,), jnp.int32),
          pltpu.VMEM((cb, h), jnp.float32),
          pltpu.VMEM((cb, h), jnp.float32),
          pltpu.SemaphoreType.DMA,
          pltpu.SemaphoreType.DMA,
          pltpu.SemaphoreType.DMA,
          pltpu.SemaphoreType.DMA,
          pltpu.VMEM_SHARED((n, h), jnp.float32),
      ],
  )
  def k(*refs):
    if use_vals:
      (vals_hbm, dst_hbm, sums_hbm, ib0, ib1, rb0, rb1,
       si0, si1, sr0, sr1, s_sh) = refs
    else:
      (dst_hbm, sums_hbm, ib0, ib1, rb0, rb1,
       si0, si1, sr0, sr1, s_sh) = refs
    cid = lax.axis_index("c")
    sid = lax.axis_index("s")
    wid = sid * NC + cid
    ibs, rbs = (ib0, ib1), (rb0, rb1)
    sis, srs = (si0, si1), (sr0, sr1)

    fill = jnp.zeros((16,), jnp.float32)
    for rb in rbs:
      for r in range(cb):
        for q in range(h // 16):
          rb[r, pl.ds(q * 16, 16)] = fill

    base_r = sid * spacing
    for zc in range(n_zchunks):
      pltpu.sync_copy(rbs[zc % 2], s_sh.at[pl.ds(base_r + zc * cb, cb)])
    plsc.subcore_barrier()

    if not use_vals:
      one = jnp.ones((16,), jnp.float32)
      for rb in rbs:
        for r in range(cb):
          for q in range(h // 16):
            rb[r, pl.ds(q * 16, 16)] = one

    for bb in range(2):
      off = wid * per_w + bb * cb
      pltpu.async_copy(dst_hbm.at[pl.ds(off, cb)], ibs[bb], sis[bb])
      if use_vals:
        pltpu.async_copy(vals_hbm.at[pl.ds(off, cb)], rbs[bb], srs[bb])

    def chunk_work(jj, bb, may_prefetch):
      off = wid * per_w + jj * cb
      pltpu.make_async_copy(
          dst_hbm.at[pl.ds(off, cb)], ibs[bb], sis[bb]).wait()
      if use_vals:
        pltpu.make_async_copy(
            vals_hbm.at[pl.ds(off, cb)], rbs[bb], srs[bb]).wait()
      pltpu.sync_copy(rbs[bb], s_sh.at[ibs[bb]], add=True)
      if may_prefetch:
        @pl.when(jj + 2 < n_chunks)
        def _():
          pltpu.async_copy(
              dst_hbm.at[pl.ds(off + 2 * cb, cb)], ibs[bb], sis[bb])
          if use_vals:
            pltpu.async_copy(
                vals_hbm.at[pl.ds(off + 2 * cb, cb)], rbs[bb], srs[bb])

    def body(g, carry):
      for bb in range(2):
        chunk_work(g * 2 + bb, bb, True)
      return carry

    lax.fori_loop(0, n_chunks // 2, body, 0)
    if n_chunks % 2:
      chunk_work(n_chunks - 1, (n_chunks - 1) % 2, False)
    plsc.subcore_barrier()

    out_base = cid * n + sid * spacing
    for zc in range(n_zchunks):
      pltpu.sync_copy(s_sh.at[pl.ds(base_r + zc * cb, cb)], rbs[zc % 2])
      pltpu.sync_copy(rbs[zc % 2], sums_hbm.at[pl.ds(out_base + zc * cb, cb)])

  if use_vals:
    out = k(vals, dst)
  else:
    out = k(dst)
  return out.reshape(NC, n, h)


def _elu(t):
  return jnp.where(t > 0, t, jnp.exp(jnp.minimum(t, 0.0)) - 1.0)


def _ln(r, gam, bet):
  mu = jnp.mean(r, axis=-1, keepdims=True)
  d = r - mu
  var = jnp.mean(d * d, axis=-1, keepdims=True)
  return d * lax.rsqrt(var + 1e-5) * gam + bet


def _edge_body(g0, g1, ea, w0a, w0b, w0c, w1, b0, b1, gam, bet, out):
  eav = ea[...]
  t = (jnp.dot(g0[...], w0a[...], preferred_element_type=jnp.float32)
       + jnp.dot(g1[...], w0b[...], preferred_element_type=jnp.float32)
       + jnp.dot(eav, w0c[...], preferred_element_type=jnp.float32)
       + b0[...])
  t = _elu(t)
  t = jnp.dot(t, w1[...], preferred_element_type=jnp.float32) + b1[...]
  out[...] = _ln(eav + t, gam[...], bet[...])


def _edge_mlp(g, ea, w0a, w0b, w0c, w1, b0, b1, gam, bet):
  e, h = ea.shape
  be = 640 if e % 640 == 0 else 512
  grid = e // be
  assert grid * be == e
  wspec = pl.BlockSpec((h, h), lambda i: (0, 0))
  vspec = pl.BlockSpec((1, h), lambda i: (0, 0))
  return pl.pallas_call(
      _edge_body,
      grid=(grid,),
      in_specs=[
          pl.BlockSpec((be, h), lambda i: (i, 0)),
          pl.BlockSpec((be, h), lambda i, g_=grid: (i + g_, 0)),
          pl.BlockSpec((be, h), lambda i: (i, 0)),
          wspec, wspec, wspec, wspec, vspec, vspec, vspec, vspec,
      ],
      out_specs=pl.BlockSpec((be, h), lambda i: (i, 0)),
      out_shape=jax.ShapeDtypeStruct((e, h), jnp.float32),
      compiler_params=pltpu.CompilerParams(
          dimension_semantics=("arbitrary",)),
  )(g, g, ea, w0a, w0b, w0c, w1, b0, b1, gam, bet)


def _node_body(final, xb, sums1, sums2, cnts, wa, wb, w1, b0, b1, gam, bet,
               wo, bo, out, *maybe_proj):
  xv = xb[...]
  s = sums1[0] + sums1[1] + sums2[0] + sums2[1]
  c = jnp.maximum(cnts[0] + cnts[1], 1.0)
  agg = s / c
  t = (jnp.dot(xv, wa[...], preferred_element_type=jnp.float32)
       + jnp.dot(agg, wb[...], preferred_element_type=jnp.float32)
       + b0[...])
  t = _elu(t)
  t = jnp.dot(t, w1[...], preferred_element_type=jnp.float32) + b1[...]
  xn = _ln(xv + t, gam[...], bet[...])
  out[...] = xn
  if final:
    proj = jnp.dot(xn, wo[...], preferred_element_type=jnp.float32) + bo[...]
    maybe_proj[0][...] = _elu(proj)


def _node_mlp(x, sums1, sums2, cnts, wa, wb, w1, b0, b1, gam, bet, wo, bo,
              final):
  n, h = x.shape
  bn = 512
  grid = pl.cdiv(n, bn)
  wspec = pl.BlockSpec((h, h), lambda i: (0, 0))
  vspec = pl.BlockSpec((1, h), lambda i: (0, 0))
  out_shape = [jax.ShapeDtypeStruct((n, h), jnp.float32)]
  out_specs = [pl.BlockSpec((bn, h), lambda i: (i, 0))]
  if final:
    out_shape.append(jax.ShapeDtypeStruct((n, h), jnp.float32))
    out_specs.append(pl.BlockSpec((bn, h), lambda i: (i, 0)))
  res = pl.pallas_call(
      functools.partial(_node_body, final),
      grid=(grid,),
      in_specs=[
          pl.BlockSpec((bn, h), lambda i: (i, 0)),
          pl.BlockSpec((NC, bn, h), lambda i: (0, i, 0)),
          pl.BlockSpec((NC, bn, h), lambda i: (0, i, 0)),
          pl.BlockSpec((NC, bn, h), lambda i: (0, i, 0)),
          wspec, wspec, wspec, vspec, vspec, vspec, vspec,
          wspec, vspec,
      ],
      out_specs=out_specs,
      out_shape=out_shape,
      compiler_params=pltpu.CompilerParams(
          dimension_semantics=("arbitrary",)),
  )(x, sums1, sums2, cnts, wa, wb, w1, b0, b1, gam, bet, wo, bo)
  return res if final else res[0]


def kernel(x, edge_index, edge_attr, edge_indices, edge_indices_f2c, clusters,
           batches, positions, lengthscales, params):
  n, h = x.shape
  ei = edge_indices[0]
  e = ei.shape[1]
  ek = e // 2
  idx_parts = [jnp.concatenate([ei[0, k * ek:(k + 1) * ek],
                                ei[1, k * ek:(k + 1) * ek]]) for k in range(2)]
  dst_parts = [ei[1, k * ek:(k + 1) * ek] for k in range(2)]

  wout_pad = jnp.zeros((h, h), jnp.float32).at[:, :params['wout'].shape[1]].set(
      params['wout'])
  bout_pad = jnp.zeros((1, h), jnp.float32).at[:, :params['bout'].shape[0]].set(
      params['bout'][None, :])

  def row(v):
    return v.reshape(1, h)

  ea_parts = [edge_attr[k * ek:(k + 1) * ek] for k in range(2)]
  out_proj = None
  n_mp = 2
  cnts = _sc_scatter_add(None, ei[1], n, h)
  for i in range(n_mp):
    w0 = params['ew0_%d' % i]
    new_ea = []
    sums_parts = []
    for k in range(2):
      g = _sc_gather(x, idx_parts[k])
      new_ea.append(
          _edge_mlp(g, ea_parts[k], w0[:h], w0[h:2 * h], w0[2 * h:],
                    params['ew1_%d' % i], row(params['eb0_%d' % i]),
                    row(params['eb1_%d' % i]), row(params['eg_%d' % i]),
                    row(params['ebt_%d' % i])))
      sums_parts.append(_sc_scatter_add(new_ea[k], dst_parts[k], n, h))
    ea_parts = new_ea
    nw0 = params['nw0_%d' % i]
    final = i == n_mp - 1
    res = _node_mlp(x, sums_parts[0], sums_parts[1], cnts, nw0[:h], nw0[h:],
                    params['nw1_%d' % i], row(params['nb0_%d' % i]),
                    row(params['nb1_%d' % i]), row(params['ng_%d' % i]),
                    row(params['nbt_%d' % i]), wout_pad, bout_pad, final)
    if final:
      x, out_proj = res
    else:
      x = res

  return (out_proj[:, :params['wout'].shape[1]], ei)


# speedup vs baseline: 3.1316x; 1.0099x over previous
import functools

import jax
import jax.numpy as jnp
from jax import lax
from jax.experimental import pallas as pl
from jax.experimental.pallas import tpu as pltpu
from jax.experimental.pallas import tpu_sc as plsc

NC = 2
NS = 16
NW = NC * NS


def _sc_gather(table, idx):
  n, h = table.shape
  b = idx.shape[0]
  per_w = b // NW
  cb = next(c for c in (80, 40, 16, 8) if per_w % c == 0)
  n_chunks = per_w // cb
  assert per_w * NW == b and n_chunks * cb == per_w

  mesh = plsc.VectorSubcoreMesh(core_axis_name="c", subcore_axis_name="s")

  assert n_chunks >= 2

  @functools.partial(
      pl.kernel,
      mesh=mesh,
      out_type=jax.ShapeDtypeStruct((b, h), jnp.float32),
      scratch_types=[
          pltpu.VMEM((cb,), jnp.int32),
          pltpu.VMEM((cb,), jnp.int32),
          pltpu.VMEM((cb, h), jnp.float32),
          pltpu.VMEM((cb, h), jnp.float32),
          pltpu.SemaphoreType.DMA,
          pltpu.SemaphoreType.DMA,
          pltpu.SemaphoreType.DMA,
          pltpu.SemaphoreType.DMA,
          pltpu.SemaphoreType.DMA,
          pltpu.SemaphoreType.DMA,
      ],
  )
  def k(table_hbm, idx_hbm, out_hbm, ib0, ib1, rb0, rb1,
        si0, si1, sg0, sg1, so0, so1):
    wid = lax.axis_index("s") * NC + lax.axis_index("c")
    base = wid * per_w
    ibs, rbs = (ib0, ib1), (rb0, rb1)
    sis, sgs, sos = (si0, si1), (sg0, sg1), (so0, so1)

    for bb in range(2):
      pltpu.async_copy(idx_hbm.at[pl.ds(base + bb * cb, cb)], ibs[bb], sis[bb])

    def chunk_work(jj, bb, may_prefetch):
      off = base + jj * cb
      @pl.when(jj >= 2)
      def _():
        pltpu.make_async_copy(
            rbs[bb], out_hbm.at[pl.ds(off - 2 * cb, cb)], sos[bb]).wait()
      pltpu.make_async_copy(
          idx_hbm.at[pl.ds(off, cb)], ibs[bb], sis[bb]).wait()
      pltpu.async_copy(table_hbm.at[ibs[bb]], rbs[bb], sgs[bb]).wait()
      pltpu.async_copy(rbs[bb], out_hbm.at[pl.ds(off, cb)], sos[bb])
      if may_prefetch:
        @pl.when(jj + 2 < n_chunks)
        def _():
          pltpu.async_copy(
              idx_hbm.at[pl.ds(off + 2 * cb, cb)], ibs[bb], sis[bb])

    def outer(g, carry):
      for bb in range(2):
        chunk_work(g * 2 + bb, bb, True)
      return carry

    lax.fori_loop(0, n_chunks // 2, outer, 0)
    if n_chunks % 2:
      chunk_work(n_chunks - 1, (n_chunks - 1) % 2, False)
    for bb in range(2):
      jj_last = n_chunks - 1 - ((n_chunks - 1 - bb) % 2)
      off = base + jj_last * cb
      pltpu.make_async_copy(
          rbs[bb], out_hbm.at[pl.ds(off, cb)], sos[bb]).wait()

  return k(table, idx)


def _sc_scatter_add(vals, dst, n, h):
  e = dst.shape[0]
  per_w = e // NW
  cb = next(c for c in (80, 40, 16, 8) if per_w % c == 0)
  n_chunks = per_w // cb
  zr = 640
  spacing = 624
  assert per_w * NW == e and n_chunks * cb == per_w
  assert spacing % 8 == 0 and spacing <= zr and spacing * (NS - 1) + zr == n
  n_zchunks = zr // cb
  use_vals = vals is not None

  mesh = plsc.VectorSubcoreMesh(core_axis_name="c", subcore_axis_name="s")

  @functools.partial(
      pl.kernel,
      mesh=mesh,
      out_type=jax.ShapeDtypeStruct((NC * n, h), jnp.float32),
      scratch_types=[
          pltpu.VMEM((cb,), jnp.int32),
          pltpu.VMEM((cb,), jnp.int32),
          pltpu.VMEM((cb, h), jnp.float32),
          pltpu.VMEM((cb, h), jnp.float32),
          pltpu.SemaphoreType.DMA,
          pltpu.SemaphoreType.DMA,
          pltpu.SemaphoreType.DMA,
          pltpu.SemaphoreType.DMA,
          pltpu.VMEM_SHARED((n, h), jnp.float32),
      ],
  )
  def k(*refs):
    if use_vals:
      (vals_hbm, dst_hbm, sums_hbm, ib0, ib1, rb0, rb1,
       si0, si1, sr0, sr1, s_sh) = refs
    else:
      (dst_hbm, sums_hbm, ib0, ib1, rb0, rb1,
       si0, si1, sr0, sr1, s_sh) = refs
    cid = lax.axis_index("c")
    sid = lax.axis_index("s")
    wid = sid * NC + cid
    ibs, rbs = (ib0, ib1), (rb0, rb1)
    sis, srs = (si0, si1), (sr0, sr1)

    fill = jnp.zeros((16,), jnp.float32)
    for rb in rbs:
      for r in range(cb):
        for q in range(h // 16):
          rb[r, pl.ds(q * 16, 16)] = fill

    base_r = sid * spacing
    for zc in range(n_zchunks):
      pltpu.sync_copy(rbs[zc % 2], s_sh.at[pl.ds(base_r + zc * cb, cb)])
    plsc.subcore_barrier()

    if not use_vals:
      one = jnp.ones((16,), jnp.float32)
      for rb in rbs:
        for r in range(cb):
          for q in range(h // 16):
            rb[r, pl.ds(q * 16, 16)] = one

    for bb in range(2):
      off = wid * per_w + bb * cb
      pltpu.async_copy(dst_hbm.at[pl.ds(off, cb)], ibs[bb], sis[bb])
      if use_vals:
        pltpu.async_copy(vals_hbm.at[pl.ds(off, cb)], rbs[bb], srs[bb])

    def chunk_work(jj, bb, may_prefetch):
      off = wid * per_w + jj * cb
      pltpu.make_async_copy(
          dst_hbm.at[pl.ds(off, cb)], ibs[bb], sis[bb]).wait()
      if use_vals:
        pltpu.make_async_copy(
            vals_hbm.at[pl.ds(off, cb)], rbs[bb], srs[bb]).wait()
      pltpu.sync_copy(rbs[bb], s_sh.at[ibs[bb]], add=True)
      if may_prefetch:
        @pl.when(jj + 2 < n_chunks)
        def _():
          pltpu.async_copy(
              dst_hbm.at[pl.ds(off + 2 * cb, cb)], ibs[bb], sis[bb])
          if use_vals:
            pltpu.async_copy(
                vals_hbm.at[pl.ds(off + 2 * cb, cb)], rbs[bb], srs[bb])

    def body(g, carry):
      for bb in range(2):
        chunk_work(g * 2 + bb, bb, True)
      return carry

    lax.fori_loop(0, n_chunks // 2, body, 0)
    if n_chunks % 2:
      chunk_work(n_chunks - 1, (n_chunks - 1) % 2, False)
    plsc.subcore_barrier()

    out_base = cid * n + sid * spacing
    for zc in range(n_zchunks):
      pltpu.sync_copy(s_sh.at[pl.ds(base_r + zc * cb, cb)], rbs[zc % 2])
      pltpu.sync_copy(rbs[zc % 2], sums_hbm.at[pl.ds(out_base + zc * cb, cb)])

  if use_vals:
    out = k(vals, dst)
  else:
    out = k(dst)
  return out.reshape(NC, n, h)


def _elu(t):
  return jnp.where(t > 0, t, jnp.exp(jnp.minimum(t, 0.0)) - 1.0)


def _ln(r, gam, bet):
  mu = jnp.mean(r, axis=-1, keepdims=True)
  d = r - mu
  var = jnp.mean(d * d, axis=-1, keepdims=True)
  return d * lax.rsqrt(var + 1e-5) * gam + bet


def _edge_body(g0, g1, ea, w0c, w1, b0, b1, gam, bet, out):
  eav = ea[...]
  t = (g0[...] + g1[...]
       + jnp.dot(eav, w0c[...], preferred_element_type=jnp.float32)
       + b0[...])
  t = _elu(t)
  t = jnp.dot(t, w1[...], preferred_element_type=jnp.float32) + b1[...]
  out[...] = _ln(eav + t, gam[...], bet[...])


def _edge_mlp(g, ea, w0c, w1, b0, b1, gam, bet):
  e, h = ea.shape
  be = 640 if e % 640 == 0 else 512
  grid = e // be
  assert grid * be == e
  wspec = pl.BlockSpec((h, h), lambda i: (0, 0))
  vspec = pl.BlockSpec((1, h), lambda i: (0, 0))
  return pl.pallas_call(
      _edge_body,
      grid=(grid,),
      in_specs=[
          pl.BlockSpec((be, h), lambda i: (i, 0)),
          pl.BlockSpec((be, h), lambda i, g_=grid: (i + g_, 0)),
          pl.BlockSpec((be, h), lambda i: (i, 0)),
          wspec, wspec, vspec, vspec, vspec, vspec,
      ],
      out_specs=pl.BlockSpec((be, h), lambda i: (i, 0)),
      out_shape=jax.ShapeDtypeStruct((e, h), jnp.float32),
      compiler_params=pltpu.CompilerParams(
          dimension_semantics=("arbitrary",)),
  )(g, g, ea, w0c, w1, b0, b1, gam, bet)


def _pre_body(xb, wab, out):
  out[...] = jnp.dot(xb[...], wab[0], preferred_element_type=jnp.float32)


def _pre_products(x, w0ab):
  n, h = x.shape
  bn = 400
  nb = n // bn
  assert nb * bn == n
  return pl.pallas_call(
      _pre_body,
      grid=(2, nb),
      in_specs=[
          pl.BlockSpec((bn, h), lambda m, i: (i, 0)),
          pl.BlockSpec((1, h, h), lambda m, i: (m, 0, 0)),
      ],
      out_specs=pl.BlockSpec((bn, h), lambda m, i, nb_=nb: (m * nb_ + i, 0)),
      out_shape=jax.ShapeDtypeStruct((2 * n, h), jnp.float32),
      compiler_params=pltpu.CompilerParams(
          dimension_semantics=("arbitrary", "arbitrary")),
  )(x, w0ab)


def _node_body(final, xb, sums1, sums2, cnts, wa, wb, w1, b0, b1, gam, bet,
               wo, bo, out, *maybe_proj):
  xv = xb[...]
  s = sums1[0] + sums1[1] + sums2[0] + sums2[1]
  c = jnp.maximum(cnts[0] + cnts[1], 1.0)
  agg = s / c
  t = (jnp.dot(xv, wa[...], preferred_element_type=jnp.float32)
       + jnp.dot(agg, wb[...], preferred_element_type=jnp.float32)
       + b0[...])
  t = _elu(t)
  t = jnp.dot(t, w1[...], preferred_element_type=jnp.float32) + b1[...]
  xn = _ln(xv + t, gam[...], bet[...])
  out[...] = xn
  if final:
    proj = jnp.dot(xn, wo[...], preferred_element_type=jnp.float32) + bo[...]
    maybe_proj[0][...] = _elu(proj)


def _node_mlp(x, sums1, sums2, cnts, wa, wb, w1, b0, b1, gam, bet, wo, bo,
              final):
  n, h = x.shape
  bn = 512
  grid = pl.cdiv(n, bn)
  wspec = pl.BlockSpec((h, h), lambda i: (0, 0))
  vspec = pl.BlockSpec((1, h), lambda i: (0, 0))
  out_shape = [jax.ShapeDtypeStruct((n, h), jnp.float32)]
  out_specs = [pl.BlockSpec((bn, h), lambda i: (i, 0))]
  if final:
    out_shape.append(jax.ShapeDtypeStruct((n, h), jnp.float32))
    out_specs.append(pl.BlockSpec((bn, h), lambda i: (i, 0)))
  res = pl.pallas_call(
      functools.partial(_node_body, final),
      grid=(grid,),
      in_specs=[
          pl.BlockSpec((bn, h), lambda i: (i, 0)),
          pl.BlockSpec((NC, bn, h), lambda i: (0, i, 0)),
          pl.BlockSpec((NC, bn, h), lambda i: (0, i, 0)),
          pl.BlockSpec((NC, bn, h), lambda i: (0, i, 0)),
          wspec, wspec, wspec, vspec, vspec, vspec, vspec,
          wspec, vspec,
      ],
      out_specs=out_specs,
      out_shape=out_shape,
      compiler_params=pltpu.CompilerParams(
          dimension_semantics=("arbitrary",)),
  )(x, sums1, sums2, cnts, wa, wb, w1, b0, b1, gam, bet, wo, bo)
  return res if final else res[0]


def kernel(x, edge_index, edge_attr, edge_indices, edge_indices_f2c, clusters,
           batches, positions, lengthscales, params):
  n, h = x.shape
  ei = edge_indices[0]
  e = ei.shape[1]
  ek = e // 2
  idx_parts = [jnp.concatenate([ei[0, k * ek:(k + 1) * ek],
                                ei[1, k * ek:(k + 1) * ek] + n])
               for k in range(2)]
  dst_parts = [ei[1, k * ek:(k + 1) * ek] for k in range(2)]

  wout_pad = jnp.zeros((h, h), jnp.float32).at[:, :params['wout'].shape[1]].set(
      params['wout'])
  bout_pad = jnp.zeros((1, h), jnp.float32).at[:, :params['bout'].shape[0]].set(
      params['bout'][None, :])

  def row(v):
    return v.reshape(1, h)

  ea_parts = [edge_attr[k * ek:(k + 1) * ek] for k in range(2)]
  out_proj = None
  n_mp = 2
  cnts = _sc_scatter_add(None, ei[1], n, h)
  for i in range(n_mp):
    w0 = params['ew0_%d' % i]
    table = _pre_products(x, jnp.stack([w0[:h], w0[h:2 * h]]))
    new_ea = []
    sums_parts = []
    for k in range(2):
      g = _sc_gather(table, idx_parts[k])
      new_ea.append(
          _edge_mlp(g, ea_parts[k], w0[2 * h:],
                    params['ew1_%d' % i], row(params['eb0_%d' % i]),
                    row(params['eb1_%d' % i]), row(params['eg_%d' % i]),
                    row(params['ebt_%d' % i])))
      sums_parts.append(_sc_scatter_add(new_ea[k], dst_parts[k], n, h))
    ea_parts = new_ea
    nw0 = params['nw0_%d' % i]
    final = i == n_mp - 1
    res = _node_mlp(x, sums_parts[0], sums_parts[1], cnts, nw0[:h], nw0[h:],
                    params['nw1_%d' % i], row(params['nb0_%d' % i]),
                    row(params['nb1_%d' % i]), row(params['ng_%d' % i]),
                    row(params['nbt_%d' % i]), wout_pad, bout_pad, final)
    if final:
      x, out_proj = res
    else:
      x = res

  return (out_proj[:, :params['wout'].shape[1]], ei)


# speedup vs baseline: 3.2442x; 1.0360x over previous
import functools

import jax
import jax.numpy as jnp
from jax import lax
from jax.experimental import pallas as pl
from jax.experimental.pallas import tpu as pltpu
from jax.experimental.pallas import tpu_sc as plsc

NC = 2
NS = 16
NW = NC * NS


def _sc_gather(table, idx):
  n, h = table.shape
  b = idx.shape[0]
  per_w = b // NW
  cb = next(c for c in (80, 40, 16, 8) if per_w % c == 0)
  n_chunks = per_w // cb
  assert per_w * NW == b and n_chunks * cb == per_w

  mesh = plsc.VectorSubcoreMesh(core_axis_name="c", subcore_axis_name="s")

  assert n_chunks >= 6

  @functools.partial(
      pl.kernel,
      mesh=mesh,
      out_type=jax.ShapeDtypeStruct((b, h), jnp.float32),
      scratch_types=(
          [pltpu.VMEM((cb,), jnp.int32)] * 4
          + [pltpu.VMEM((cb, h), jnp.float32)] * 2
          + [pltpu.SemaphoreType.DMA] * 8
      ),
  )
  def k(table_hbm, idx_hbm, out_hbm, ib0, ib1, ib2, ib3, rb0, rb1,
        si0, si1, si2, si3, sg0, sg1, so0, so1):
    wid = lax.axis_index("s") * NC + lax.axis_index("c")
    base = wid * per_w
    ibs, rbs = (ib0, ib1, ib2, ib3), (rb0, rb1)
    sis, sgs, sos = (si0, si1, si2, si3), (sg0, sg1), (so0, so1)

    def idx_fetch(jj, b4):
      pltpu.async_copy(idx_hbm.at[pl.ds(base + jj * cb, cb)],
                       ibs[b4], sis[b4])

    def gather_start(jj, b2, b4):
      pltpu.make_async_copy(idx_hbm.at[pl.ds(base + jj * cb, cb)],
                            ibs[b4], sis[b4]).wait()
      pltpu.async_copy(table_hbm.at[ibs[b4]], rbs[b2], sgs[b2])

    def out_start(jj, b2):
      pltpu.make_async_copy(table_hbm.at[pl.ds(0, cb)], rbs[b2],
                            sgs[b2]).wait()
      pltpu.async_copy(rbs[b2], out_hbm.at[pl.ds(base + jj * cb, cb)],
                       sos[b2])

    def out_drain(jj, b2):
      pltpu.make_async_copy(rbs[b2], out_hbm.at[pl.ds(base + jj * cb, cb)],
                            sos[b2]).wait()

    for jj in range(4):
      idx_fetch(jj, jj % 4)
    gather_start(0, 0, 0)
    gather_start(1, 1, 1)
    out_start(0, 0)

    def body(g, carry):
      for u in range(4):
        jj = g * 4 + 2 + u
        b2, b4 = u % 2, (2 + u) % 4
        out_drain(jj - 2, b2)
        gather_start(jj, b2, b4)
        out_start(jj - 1, 1 - b2)
        idx_fetch(jj + 2, u % 4)
      return carry

    n_mid = (n_chunks - 4) // 4
    lax.fori_loop(0, n_mid, body, 0)

    for jj in range(2 + n_mid * 4, n_chunks):
      b2, b4 = jj % 2, jj % 4
      out_drain(jj - 2, b2)
      gather_start(jj, b2, b4)
      out_start(jj - 1, 1 - b2)
      if jj + 2 < n_chunks:
        idx_fetch(jj + 2, (jj + 2) % 4)

    last = n_chunks - 1
    out_start(last, last % 2)
    out_drain(last - 1, (last - 1) % 2)
    out_drain(last, last % 2)

  return k(table, idx)


def _sc_scatter_add(vals, dst, n, h):
  e = dst.shape[0]
  per_w = e // NW
  cb = next(c for c in (80, 40, 16, 8) if per_w % c == 0)
  n_chunks = per_w // cb
  zr = 640
  spacing = 624
  assert per_w * NW == e and n_chunks * cb == per_w
  assert spacing % 8 == 0 and spacing <= zr and spacing * (NS - 1) + zr == n
  n_zchunks = zr // cb
  use_vals = vals is not None

  mesh = plsc.VectorSubcoreMesh(core_axis_name="c", subcore_axis_name="s")

  @functools.partial(
      pl.kernel,
      mesh=mesh,
      out_type=jax.ShapeDtypeStruct((NC * n, h), jnp.float32),
      scratch_types=[
          pltpu.VMEM((cb,), jnp.int32),
          pltpu.VMEM((cb,), jnp.int32),
          pltpu.VMEM((cb, h), jnp.float32),
          pltpu.VMEM((cb, h), jnp.float32),
          pltpu.SemaphoreType.DMA,
          pltpu.SemaphoreType.DMA,
          pltpu.SemaphoreType.DMA,
          pltpu.SemaphoreType.DMA,
          pltpu.VMEM_SHARED((n, h), jnp.float32),
      ],
  )
  def k(*refs):
    if use_vals:
      (vals_hbm, dst_hbm, sums_hbm, ib0, ib1, rb0, rb1,
       si0, si1, sr0, sr1, s_sh) = refs
    else:
      (dst_hbm, sums_hbm, ib0, ib1, rb0, rb1,
       si0, si1, sr0, sr1, s_sh) = refs
    cid = lax.axis_index("c")
    sid = lax.axis_index("s")
    wid = sid * NC + cid
    ibs, rbs = (ib0, ib1), (rb0, rb1)
    sis, srs = (si0, si1), (sr0, sr1)

    fill = jnp.zeros((16,), jnp.float32)
    for rb in rbs:
      for r in range(cb):
        for q in range(h // 16):
          rb[r, pl.ds(q * 16, 16)] = fill

    base_r = sid * spacing
    for zc in range(n_zchunks):
      pltpu.sync_copy(rbs[zc % 2], s_sh.at[pl.ds(base_r + zc * cb, cb)])
    plsc.subcore_barrier()

    if not use_vals:
      one = jnp.ones((16,), jnp.float32)
      for rb in rbs:
        for r in range(cb):
          for q in range(h // 16):
            rb[r, pl.ds(q * 16, 16)] = one

    for bb in range(2):
      off = wid * per_w + bb * cb
      pltpu.async_copy(dst_hbm.at[pl.ds(off, cb)], ibs[bb], sis[bb])
      if use_vals:
        pltpu.async_copy(vals_hbm.at[pl.ds(off, cb)], rbs[bb], srs[bb])

    def chunk_work(jj, bb, may_prefetch):
      off = wid * per_w + jj * cb
      pltpu.make_async_copy(
          dst_hbm.at[pl.ds(off, cb)], ibs[bb], sis[bb]).wait()
      if use_vals:
        pltpu.make_async_copy(
            vals_hbm.at[pl.ds(off, cb)], rbs[bb], srs[bb]).wait()
      pltpu.sync_copy(rbs[bb], s_sh.at[ibs[bb]], add=True)
      if may_prefetch:
        @pl.when(jj + 2 < n_chunks)
        def _():
          pltpu.async_copy(
              dst_hbm.at[pl.ds(off + 2 * cb, cb)], ibs[bb], sis[bb])
          if use_vals:
            pltpu.async_copy(
                vals_hbm.at[pl.ds(off + 2 * cb, cb)], rbs[bb], srs[bb])

    def body(g, carry):
      for bb in range(2):
        chunk_work(g * 2 + bb, bb, True)
      return carry

    lax.fori_loop(0, n_chunks // 2, body, 0)
    if n_chunks % 2:
      chunk_work(n_chunks - 1, (n_chunks - 1) % 2, False)
    plsc.subcore_barrier()

    out_base = cid * n + sid * spacing
    for zc in range(n_zchunks):
      pltpu.sync_copy(s_sh.at[pl.ds(base_r + zc * cb, cb)], rbs[zc % 2])
      pltpu.sync_copy(rbs[zc % 2], sums_hbm.at[pl.ds(out_base + zc * cb, cb)])

  if use_vals:
    out = k(vals, dst)
  else:
    out = k(dst)
  return out.reshape(NC, n, h)


def _elu(t):
  return jnp.where(t > 0, t, jnp.exp(jnp.minimum(t, 0.0)) - 1.0)


def _ln(r, gam, bet):
  mu = jnp.mean(r, axis=-1, keepdims=True)
  d = r - mu
  var = jnp.mean(d * d, axis=-1, keepdims=True)
  return d * lax.rsqrt(var + 1e-5) * gam + bet


def _edge_body(g0, g1, ea, w0c, w1, b0, b1, gam, bet, out):
  eav = ea[...]
  t = (g0[...] + g1[...]
       + jnp.dot(eav, w0c[...], preferred_element_type=jnp.float32)
       + b0[...])
  t = _elu(t)
  t = jnp.dot(t, w1[...], preferred_element_type=jnp.float32) + b1[...]
  out[...] = _ln(eav + t, gam[...], bet[...])


def _edge_mlp(g, ea, w0c, w1, b0, b1, gam, bet):
  e, h = ea.shape
  be = 640 if e % 640 == 0 else 512
  grid = e // be
  assert grid * be == e
  wspec = pl.BlockSpec((h, h), lambda i: (0, 0))
  vspec = pl.BlockSpec((1, h), lambda i: (0, 0))
  return pl.pallas_call(
      _edge_body,
      grid=(grid,),
      in_specs=[
          pl.BlockSpec((be, h), lambda i: (i, 0)),
          pl.BlockSpec((be, h), lambda i, g_=grid: (i + g_, 0)),
          pl.BlockSpec((be, h), lambda i: (i, 0)),
          wspec, wspec, vspec, vspec, vspec, vspec,
      ],
      out_specs=pl.BlockSpec((be, h), lambda i: (i, 0)),
      out_shape=jax.ShapeDtypeStruct((e, h), jnp.float32),
      compiler_params=pltpu.CompilerParams(
          dimension_semantics=("arbitrary",)),
  )(g, g, ea, w0c, w1, b0, b1, gam, bet)


def _pre_body(xb, wab, out):
  out[...] = jnp.dot(xb[...], wab[0], preferred_element_type=jnp.float32)


def _pre_products(x, w0ab):
  n, h = x.shape
  bn = 400
  nb = n // bn
  assert nb * bn == n
  return pl.pallas_call(
      _pre_body,
      grid=(2, nb),
      in_specs=[
          pl.BlockSpec((bn, h), lambda m, i: (i, 0)),
          pl.BlockSpec((1, h, h), lambda m, i: (m, 0, 0)),
      ],
      out_specs=pl.BlockSpec((bn, h), lambda m, i, nb_=nb: (m * nb_ + i, 0)),
      out_shape=jax.ShapeDtypeStruct((2 * n, h), jnp.float32),
      compiler_params=pltpu.CompilerParams(
          dimension_semantics=("arbitrary", "arbitrary")),
  )(x, w0ab)


def _node_body(final, xb, sums1, sums2, cnts, wa, wb, w1, b0, b1, gam, bet,
               wo, bo, out, *maybe_proj):
  xv = xb[...]
  s = sums1[0] + sums1[1] + sums2[0] + sums2[1]
  c = jnp.maximum(cnts[0] + cnts[1], 1.0)
  agg = s / c
  t = (jnp.dot(xv, wa[...], preferred_element_type=jnp.float32)
       + jnp.dot(agg, wb[...], preferred_element_type=jnp.float32)
       + b0[...])
  t = _elu(t)
  t = jnp.dot(t, w1[...], preferred_element_type=jnp.float32) + b1[...]
  xn = _ln(xv + t, gam[...], bet[...])
  out[...] = xn
  if final:
    proj = jnp.dot(xn, wo[...], preferred_element_type=jnp.float32) + bo[...]
    maybe_proj[0][...] = _elu(proj)


def _node_mlp(x, sums1, sums2, cnts, wa, wb, w1, b0, b1, gam, bet, wo, bo,
              final):
  n, h = x.shape
  bn = 512
  grid = pl.cdiv(n, bn)
  wspec = pl.BlockSpec((h, h), lambda i: (0, 0))
  vspec = pl.BlockSpec((1, h), lambda i: (0, 0))
  out_shape = [jax.ShapeDtypeStruct((n, h), jnp.float32)]
  out_specs = [pl.BlockSpec((bn, h), lambda i: (i, 0))]
  if final:
    out_shape.append(jax.ShapeDtypeStruct((n, h), jnp.float32))
    out_specs.append(pl.BlockSpec((bn, h), lambda i: (i, 0)))
  res = pl.pallas_call(
      functools.partial(_node_body, final),
      grid=(grid,),
      in_specs=[
          pl.BlockSpec((bn, h), lambda i: (i, 0)),
          pl.BlockSpec((NC, bn, h), lambda i: (0, i, 0)),
          pl.BlockSpec((NC, bn, h), lambda i: (0, i, 0)),
          pl.BlockSpec((NC, bn, h), lambda i: (0, i, 0)),
          wspec, wspec, wspec, vspec, vspec, vspec, vspec,
          wspec, vspec,
      ],
      out_specs=out_specs,
      out_shape=out_shape,
      compiler_params=pltpu.CompilerParams(
          dimension_semantics=("arbitrary",)),
  )(x, sums1, sums2, cnts, wa, wb, w1, b0, b1, gam, bet, wo, bo)
  return res if final else res[0]


def kernel(x, edge_index, edge_attr, edge_indices, edge_indices_f2c, clusters,
           batches, positions, lengthscales, params):
  n, h = x.shape
  ei = edge_indices[0]
  e = ei.shape[1]
  ek = e // 2
  idx_parts = [jnp.concatenate([ei[0, k * ek:(k + 1) * ek],
                                ei[1, k * ek:(k + 1) * ek] + n])
               for k in range(2)]
  dst_parts = [ei[1, k * ek:(k + 1) * ek] for k in range(2)]

  wout_pad = jnp.zeros((h, h), jnp.float32).at[:, :params['wout'].shape[1]].set(
      params['wout'])
  bout_pad = jnp.zeros((1, h), jnp.float32).at[:, :params['bout'].shape[0]].set(
      params['bout'][None, :])

  def row(v):
    return v.reshape(1, h)

  ea_parts = [edge_attr[k * ek:(k + 1) * ek] for k in range(2)]
  out_proj = None
  n_mp = 2
  cnts = _sc_scatter_add(None, ei[1], n, h)
  for i in range(n_mp):
    w0 = params['ew0_%d' % i]
    table = _pre_products(x, jnp.stack([w0[:h], w0[h:2 * h]]))
    new_ea = []
    sums_parts = []
    for k in range(2):
      g = _sc_gather(table, idx_parts[k])
      new_ea.append(
          _edge_mlp(g, ea_parts[k], w0[2 * h:],
                    params['ew1_%d' % i], row(params['eb0_%d' % i]),
                    row(params['eb1_%d' % i]), row(params['eg_%d' % i]),
                    row(params['ebt_%d' % i])))
      sums_parts.append(_sc_scatter_add(new_ea[k], dst_parts[k], n, h))
    ea_parts = new_ea
    nw0 = params['nw0_%d' % i]
    final = i == n_mp - 1
    res = _node_mlp(x, sums_parts[0], sums_parts[1], cnts, nw0[:h], nw0[h:],
                    params['nw1_%d' % i], row(params['nb0_%d' % i]),
                    row(params['nb1_%d' % i]), row(params['ng_%d' % i]),
                    row(params['nbt_%d' % i]), wout_pad, bout_pad, final)
    if final:
      x, out_proj = res
    else:
      x = res

  return (out_proj[:, :params['wout'].shape[1]], ei)


# speedup vs baseline: 3.2558x; 1.0036x over previous
import functools

import jax
import jax.numpy as jnp
from jax import lax
from jax.experimental import pallas as pl
from jax.experimental.pallas import tpu as pltpu
from jax.experimental.pallas import tpu_sc as plsc

NC = 2
NS = 16
NW = NC * NS


def _sc_gather(table, idx):
  n, h = table.shape
  b = idx.shape[0]
  per_w = b // NW
  cb = next(c for c in (80, 40, 16, 8) if per_w % c == 0)
  n_chunks = per_w // cb
  assert per_w * NW == b and n_chunks * cb == per_w

  mesh = plsc.VectorSubcoreMesh(core_axis_name="c", subcore_axis_name="s")

  assert n_chunks >= 6

  @functools.partial(
      pl.kernel,
      mesh=mesh,
      out_type=jax.ShapeDtypeStruct((b, h), jnp.float32),
      scratch_types=(
          [pltpu.VMEM((cb,), jnp.int32)] * 4
          + [pltpu.VMEM((cb, h), jnp.float32)] * 2
          + [pltpu.SemaphoreType.DMA] * 8
      ),
  )
  def k(table_hbm, idx_hbm, out_hbm, ib0, ib1, ib2, ib3, rb0, rb1,
        si0, si1, si2, si3, sg0, sg1, so0, so1):
    wid = lax.axis_index("s") * NC + lax.axis_index("c")
    base = wid * per_w
    ibs, rbs = (ib0, ib1, ib2, ib3), (rb0, rb1)
    sis, sgs, sos = (si0, si1, si2, si3), (sg0, sg1), (so0, so1)

    def idx_fetch(jj, b4):
      pltpu.async_copy(idx_hbm.at[pl.ds(base + jj * cb, cb)],
                       ibs[b4], sis[b4])

    def gather_start(jj, b2, b4):
      pltpu.make_async_copy(idx_hbm.at[pl.ds(base + jj * cb, cb)],
                            ibs[b4], sis[b4]).wait()
      pltpu.async_copy(table_hbm.at[ibs[b4]], rbs[b2], sgs[b2])

    def out_start(jj, b2):
      pltpu.make_async_copy(table_hbm.at[pl.ds(0, cb)], rbs[b2],
                            sgs[b2]).wait()
      pltpu.async_copy(rbs[b2], out_hbm.at[pl.ds(base + jj * cb, cb)],
                       sos[b2])

    def out_drain(jj, b2):
      pltpu.make_async_copy(rbs[b2], out_hbm.at[pl.ds(base + jj * cb, cb)],
                            sos[b2]).wait()

    for jj in range(4):
      idx_fetch(jj, jj % 4)
    gather_start(0, 0, 0)
    gather_start(1, 1, 1)
    out_start(0, 0)

    def body(g, carry):
      for u in range(4):
        jj = g * 4 + 2 + u
        b2, b4 = u % 2, (2 + u) % 4
        out_drain(jj - 2, b2)
        gather_start(jj, b2, b4)
        out_start(jj - 1, 1 - b2)
        idx_fetch(jj + 2, u % 4)
      return carry

    n_mid = (n_chunks - 4) // 4
    lax.fori_loop(0, n_mid, body, 0)

    for jj in range(2 + n_mid * 4, n_chunks):
      b2, b4 = jj % 2, jj % 4
      out_drain(jj - 2, b2)
      gather_start(jj, b2, b4)
      out_start(jj - 1, 1 - b2)
      if jj + 2 < n_chunks:
        idx_fetch(jj + 2, (jj + 2) % 4)

    last = n_chunks - 1
    out_start(last, last % 2)
    out_drain(last - 1, (last - 1) % 2)
    out_drain(last, last % 2)

  return k(table, idx)


def _sc_scatter_add(vals, dst, n, h):
  e = dst.shape[0]
  per_w = e // NW
  cb = next(c for c in (80, 40, 16, 8) if per_w % c == 0)
  n_chunks = per_w // cb
  zr = 640
  spacing = 624
  assert per_w * NW == e and n_chunks * cb == per_w
  assert spacing % 8 == 0 and spacing <= zr and spacing * (NS - 1) + zr == n
  n_zchunks = zr // cb
  use_vals = vals is not None

  mesh = plsc.VectorSubcoreMesh(core_axis_name="c", subcore_axis_name="s")

  @functools.partial(
      pl.kernel,
      mesh=mesh,
      out_type=jax.ShapeDtypeStruct((NC * n, h), jnp.float32),
      scratch_types=[
          pltpu.VMEM((cb,), jnp.int32),
          pltpu.VMEM((cb,), jnp.int32),
          pltpu.VMEM((cb, h), jnp.float32),
          pltpu.VMEM((cb, h), jnp.float32),
          pltpu.SemaphoreType.DMA,
          pltpu.SemaphoreType.DMA,
          pltpu.SemaphoreType.DMA,
          pltpu.SemaphoreType.DMA,
          pltpu.VMEM_SHARED((n, h), jnp.float32),
      ],
  )
  def k(*refs):
    if use_vals:
      (vals_hbm, dst_hbm, sums_hbm, ib0, ib1, rb0, rb1,
       si0, si1, sr0, sr1, s_sh) = refs
    else:
      (dst_hbm, sums_hbm, ib0, ib1, rb0, rb1,
       si0, si1, sr0, sr1, s_sh) = refs
    cid = lax.axis_index("c")
    sid = lax.axis_index("s")
    wid = sid * NC + cid
    ibs, rbs = (ib0, ib1), (rb0, rb1)
    sis, srs = (si0, si1), (sr0, sr1)

    fill = jnp.zeros((16,), jnp.float32)
    for rb in rbs:
      for r in range(cb):
        for q in range(h // 16):
          rb[r, pl.ds(q * 16, 16)] = fill

    base_r = sid * spacing
    for zc in range(n_zchunks):
      pltpu.sync_copy(rbs[zc % 2], s_sh.at[pl.ds(base_r + zc * cb, cb)])
    plsc.subcore_barrier()

    if not use_vals:
      one = jnp.ones((16,), jnp.float32)
      for rb in rbs:
        for r in range(cb):
          for q in range(h // 16):
            rb[r, pl.ds(q * 16, 16)] = one

    for bb in range(2):
      off = wid * per_w + bb * cb
      pltpu.async_copy(dst_hbm.at[pl.ds(off, cb)], ibs[bb], sis[bb])
      if use_vals:
        pltpu.async_copy(vals_hbm.at[pl.ds(off, cb)], rbs[bb], srs[bb])

    def chunk_work(jj, bb, may_prefetch):
      off = wid * per_w + jj * cb
      pltpu.make_async_copy(
          dst_hbm.at[pl.ds(off, cb)], ibs[bb], sis[bb]).wait()
      if use_vals:
        pltpu.make_async_copy(
            vals_hbm.at[pl.ds(off, cb)], rbs[bb], srs[bb]).wait()
      pltpu.sync_copy(rbs[bb], s_sh.at[ibs[bb]], add=True)
      if may_prefetch:
        @pl.when(jj + 2 < n_chunks)
        def _():
          pltpu.async_copy(
              dst_hbm.at[pl.ds(off + 2 * cb, cb)], ibs[bb], sis[bb])
          if use_vals:
            pltpu.async_copy(
                vals_hbm.at[pl.ds(off + 2 * cb, cb)], rbs[bb], srs[bb])

    def body(g, carry):
      for bb in range(2):
        chunk_work(g * 2 + bb, bb, True)
      return carry

    lax.fori_loop(0, n_chunks // 2, body, 0)
    if n_chunks % 2:
      chunk_work(n_chunks - 1, (n_chunks - 1) % 2, False)
    plsc.subcore_barrier()

    out_base = cid * n + sid * spacing
    for zc in range(n_zchunks):
      pltpu.sync_copy(s_sh.at[pl.ds(base_r + zc * cb, cb)], rbs[zc % 2])
      pltpu.sync_copy(rbs[zc % 2], sums_hbm.at[pl.ds(out_base + zc * cb, cb)])

  if use_vals:
    out = k(vals, dst)
  else:
    out = k(dst)
  return out.reshape(NC, n, h)


def _elu(t):
  return jnp.where(t > 0, t, jnp.exp(jnp.minimum(t, 0.0)) - 1.0)


def _ln(r, gam, bet):
  mu = jnp.mean(r, axis=-1, keepdims=True)
  d = r - mu
  var = jnp.mean(d * d, axis=-1, keepdims=True)
  return d * lax.rsqrt(var + 1e-5) * gam + bet


def _edge_body(g0, g1, ea, w0c, w1, b0, b1, gam, bet, out):
  eav = ea[...]
  t = (g0[...] + g1[...]
       + jnp.dot(eav, w0c[...], preferred_element_type=jnp.float32)
       + b0[...])
  t = _elu(t)
  t = jnp.dot(t, w1[...], preferred_element_type=jnp.float32) + b1[...]
  out[...] = _ln(eav + t, gam[...], bet[...])


def _edge_mlp(g, ea, w0c, w1, b0, b1, gam, bet):
  e, h = ea.shape
  be = 640 if e % 640 == 0 else 512
  grid = e // be
  assert grid * be == e
  wspec = pl.BlockSpec((h, h), lambda i: (0, 0))
  vspec = pl.BlockSpec((1, h), lambda i: (0, 0))
  return pl.pallas_call(
      _edge_body,
      grid=(grid,),
      in_specs=[
          pl.BlockSpec((be, h), lambda i: (i, 0)),
          pl.BlockSpec((be, h), lambda i, g_=grid: (i + g_, 0)),
          pl.BlockSpec((be, h), lambda i: (i, 0)),
          wspec, wspec, vspec, vspec, vspec, vspec,
      ],
      out_specs=pl.BlockSpec((be, h), lambda i: (i, 0)),
      out_shape=jax.ShapeDtypeStruct((e, h), jnp.float32),
      compiler_params=pltpu.CompilerParams(
          dimension_semantics=("arbitrary",)),
  )(g, g, ea, w0c, w1, b0, b1, gam, bet)


def _pre_body(xb, wab, out):
  out[...] = jnp.dot(xb[...], wab[0], preferred_element_type=jnp.float32)


def _pre_products(x, w0ab):
  n, h = x.shape
  bn = 400
  nb = n // bn
  assert nb * bn == n
  return pl.pallas_call(
      _pre_body,
      grid=(2, nb),
      in_specs=[
          pl.BlockSpec((bn, h), lambda m, i: (i, 0)),
          pl.BlockSpec((1, h, h), lambda m, i: (m, 0, 0)),
      ],
      out_specs=pl.BlockSpec((bn, h), lambda m, i, nb_=nb: (m * nb_ + i, 0)),
      out_shape=jax.ShapeDtypeStruct((2 * n, h), jnp.float32),
      compiler_params=pltpu.CompilerParams(
          dimension_semantics=("arbitrary", "arbitrary")),
  )(x, w0ab)


def _node_body(final, xb, sums1, sums2, cnts, wa, wb, w1, b0, b1, gam, bet,
               wo, bo, wnext, out, *extra):
  xv = xb[...]
  s = sums1[0] + sums1[1] + sums2[0] + sums2[1]
  c = jnp.maximum(cnts[0] + cnts[1], 1.0)
  agg = s / c
  t = (jnp.dot(xv, wa[...], preferred_element_type=jnp.float32)
       + jnp.dot(agg, wb[...], preferred_element_type=jnp.float32)
       + b0[...])
  t = _elu(t)
  t = jnp.dot(t, w1[...], preferred_element_type=jnp.float32) + b1[...]
  xn = _ln(xv + t, gam[...], bet[...])
  out[...] = xn
  if final:
    proj = jnp.dot(xn, wo[...], preferred_element_type=jnp.float32) + bo[...]
    extra[0][...] = _elu(proj)
  else:
    wn = wnext[...]
    extra[0][...] = jnp.dot(xn, wn[0], preferred_element_type=jnp.float32)
    extra[1][...] = jnp.dot(xn, wn[1], preferred_element_type=jnp.float32)


def _node_mlp(x, sums1, sums2, cnts, wa, wb, w1, b0, b1, gam, bet, wo, bo,
              wnext, final):
  n, h = x.shape
  bn = 512
  grid = pl.cdiv(n, bn)
  wspec = pl.BlockSpec((h, h), lambda i: (0, 0))
  vspec = pl.BlockSpec((1, h), lambda i: (0, 0))
  out_shape = [jax.ShapeDtypeStruct((n, h), jnp.float32)]
  out_specs = [pl.BlockSpec((bn, h), lambda i: (i, 0))]
  n_extra = 1 if final else 2
  for _ in range(n_extra):
    out_shape.append(jax.ShapeDtypeStruct((n, h), jnp.float32))
    out_specs.append(pl.BlockSpec((bn, h), lambda i: (i, 0)))
  res = pl.pallas_call(
      functools.partial(_node_body, final),
      grid=(grid,),
      in_specs=[
          pl.BlockSpec((bn, h), lambda i: (i, 0)),
          pl.BlockSpec((NC, bn, h), lambda i: (0, i, 0)),
          pl.BlockSpec((NC, bn, h), lambda i: (0, i, 0)),
          pl.BlockSpec((NC, bn, h), lambda i: (0, i, 0)),
          wspec, wspec, wspec, vspec, vspec, vspec, vspec,
          wspec, vspec,
          pl.BlockSpec((2, h, h), lambda i: (0, 0, 0)),
      ],
      out_specs=out_specs,
      out_shape=out_shape,
      compiler_params=pltpu.CompilerParams(
          dimension_semantics=("arbitrary",)),
  )(x, sums1, sums2, cnts, wa, wb, w1, b0, b1, gam, bet, wo, bo, wnext)
  return res


def kernel(x, edge_index, edge_attr, edge_indices, edge_indices_f2c, clusters,
           batches, positions, lengthscales, params):
  n, h = x.shape
  ei = edge_indices[0]
  e = ei.shape[1]
  ek = e // 2
  idx_parts = [jnp.concatenate([ei[0, k * ek:(k + 1) * ek],
                                ei[1, k * ek:(k + 1) * ek] + n])
               for k in range(2)]
  dst_parts = [ei[1, k * ek:(k + 1) * ek] for k in range(2)]

  wout_pad = jnp.zeros((h, h), jnp.float32).at[:, :params['wout'].shape[1]].set(
      params['wout'])
  bout_pad = jnp.zeros((1, h), jnp.float32).at[:, :params['bout'].shape[0]].set(
      params['bout'][None, :])

  def row(v):
    return v.reshape(1, h)

  ea_parts = [edge_attr[k * ek:(k + 1) * ek] for k in range(2)]
  out_proj = None
  n_mp = 2
  w0_stacks = [jnp.stack([params['ew0_%d' % i][:h],
                          params['ew0_%d' % i][h:2 * h]]) for i in range(n_mp)]
  cnts = None
  table = _pre_products(x, w0_stacks[0])
  for i in range(n_mp):
    w0 = params['ew0_%d' % i]
    new_ea = []
    sums_parts = []
    for k in range(2):
      g = _sc_gather(table, idx_parts[k])
      if cnts is None:
        dst_c = jnp.where(g[0, 0] > jnp.float32(3e38), ei[1] ^ 1, ei[1])
        cnts = _sc_scatter_add(None, dst_c, n, h)
      new_ea.append(
          _edge_mlp(g, ea_parts[k], w0[2 * h:],
                    params['ew1_%d' % i], row(params['eb0_%d' % i]),
                    row(params['eb1_%d' % i]), row(params['eg_%d' % i]),
                    row(params['ebt_%d' % i])))
      sums_parts.append(_sc_scatter_add(new_ea[k], dst_parts[k], n, h))
    ea_parts = new_ea
    nw0 = params['nw0_%d' % i]
    final = i == n_mp - 1
    wnext = w0_stacks[0] if final else w0_stacks[i + 1]
    res = _node_mlp(x, sums_parts[0], sums_parts[1], cnts, nw0[:h], nw0[h:],
                    params['nw1_%d' % i], row(params['nb0_%d' % i]),
                    row(params['nb1_%d' % i]), row(params['ng_%d' % i]),
                    row(params['nbt_%d' % i]), wout_pad, bout_pad, wnext,
                    final)
    if final:
      x, out_proj = res
    else:
      x, p_next, q_next = res
      table = jnp.concatenate([p_next, q_next], axis=0)

  return (out_proj[:, :params['wout'].shape[1]], ei)


# speedup vs baseline: 3.8706x; 1.1888x over previous
import functools

import jax
import jax.numpy as jnp
from jax import lax
from jax.experimental import pallas as pl
from jax.experimental.pallas import tpu as pltpu
from jax.experimental.pallas import tpu_sc as plsc

NC = 2
NS = 16
NW = NC * NS


def _sc_gather(table, idx):
  n, h = table.shape
  b = idx.shape[0]
  per_w = b // NW
  cb = next(c for c in (80, 40, 16, 8) if per_w % c == 0)
  n_chunks = per_w // cb
  assert per_w * NW == b and n_chunks * cb == per_w

  mesh = plsc.VectorSubcoreMesh(core_axis_name="c", subcore_axis_name="s")

  assert n_chunks >= 6

  @functools.partial(
      pl.kernel,
      mesh=mesh,
      out_type=jax.ShapeDtypeStruct((b, h), jnp.float32),
      scratch_types=(
          [pltpu.VMEM((cb,), jnp.int32)] * 4
          + [pltpu.VMEM((cb, h), jnp.float32)] * 2
          + [pltpu.SemaphoreType.DMA] * 8
      ),
  )
  def k(table_hbm, idx_hbm, out_hbm, ib0, ib1, ib2, ib3, rb0, rb1,
        si0, si1, si2, si3, sg0, sg1, so0, so1):
    wid = lax.axis_index("s") * NC + lax.axis_index("c")
    base = wid * per_w
    ibs, rbs = (ib0, ib1, ib2, ib3), (rb0, rb1)
    sis, sgs, sos = (si0, si1, si2, si3), (sg0, sg1), (so0, so1)

    def idx_fetch(jj, b4):
      pltpu.async_copy(idx_hbm.at[pl.ds(base + jj * cb, cb)],
                       ibs[b4], sis[b4])

    def gather_start(jj, b2, b4):
      pltpu.make_async_copy(idx_hbm.at[pl.ds(base + jj * cb, cb)],
                            ibs[b4], sis[b4]).wait()
      pltpu.async_copy(table_hbm.at[ibs[b4]], rbs[b2], sgs[b2])

    def out_start(jj, b2):
      pltpu.make_async_copy(table_hbm.at[pl.ds(0, cb)], rbs[b2],
                            sgs[b2]).wait()
      pltpu.async_copy(rbs[b2], out_hbm.at[pl.ds(base + jj * cb, cb)],
                       sos[b2])

    def out_drain(jj, b2):
      pltpu.make_async_copy(rbs[b2], out_hbm.at[pl.ds(base + jj * cb, cb)],
                            sos[b2]).wait()

    for jj in range(4):
      idx_fetch(jj, jj % 4)
    gather_start(0, 0, 0)
    gather_start(1, 1, 1)
    out_start(0, 0)

    def body(g, carry):
      for u in range(4):
        jj = g * 4 + 2 + u
        b2, b4 = u % 2, (2 + u) % 4
        out_drain(jj - 2, b2)
        gather_start(jj, b2, b4)
        out_start(jj - 1, 1 - b2)
        idx_fetch(jj + 2, u % 4)
      return carry

    n_mid = (n_chunks - 4) // 4
    lax.fori_loop(0, n_mid, body, 0)

    for jj in range(2 + n_mid * 4, n_chunks):
      b2, b4 = jj % 2, jj % 4
      out_drain(jj - 2, b2)
      gather_start(jj, b2, b4)
      out_start(jj - 1, 1 - b2)
      if jj + 2 < n_chunks:
        idx_fetch(jj + 2, (jj + 2) % 4)

    last = n_chunks - 1
    out_start(last, last % 2)
    out_drain(last - 1, (last - 1) % 2)
    out_drain(last, last % 2)

  return k(table, idx)


def _sc_scatter_add(vals, dst, n, h):
  e = dst.shape[0]
  per_w = e // NW
  cb = next(c for c in (80, 40, 16, 8) if per_w % c == 0)
  n_chunks = per_w // cb
  zr = 640
  spacing = 624
  assert per_w * NW == e and n_chunks * cb == per_w
  assert spacing % 8 == 0 and spacing <= zr and spacing * (NS - 1) + zr == n
  n_zchunks = zr // cb
  use_vals = vals is not None

  mesh = plsc.VectorSubcoreMesh(core_axis_name="c", subcore_axis_name="s")

  @functools.partial(
      pl.kernel,
      mesh=mesh,
      out_type=jax.ShapeDtypeStruct((NC * n, h), jnp.float32),
      scratch_types=[
          pltpu.VMEM((cb,), jnp.int32),
          pltpu.VMEM((cb,), jnp.int32),
          pltpu.VMEM((cb, h), jnp.float32),
          pltpu.VMEM((cb, h), jnp.float32),
          pltpu.SemaphoreType.DMA,
          pltpu.SemaphoreType.DMA,
          pltpu.SemaphoreType.DMA,
          pltpu.SemaphoreType.DMA,
          pltpu.VMEM_SHARED((n, h), jnp.float32),
      ],
  )
  def k(*refs):
    if use_vals:
      (vals_hbm, dst_hbm, sums_hbm, ib0, ib1, rb0, rb1,
       si0, si1, sr0, sr1, s_sh) = refs
    else:
      (dst_hbm, sums_hbm, ib0, ib1, rb0, rb1,
       si0, si1, sr0, sr1, s_sh) = refs
    cid = lax.axis_index("c")
    sid = lax.axis_index("s")
    wid = sid * NC + cid
    ibs, rbs = (ib0, ib1), (rb0, rb1)
    sis, srs = (si0, si1), (sr0, sr1)

    fill = jnp.zeros((16,), jnp.float32)
    for rb in rbs:
      for r in range(cb):
        for q in range(h // 16):
          rb[r, pl.ds(q * 16, 16)] = fill

    base_r = sid * spacing
    for zc in range(n_zchunks):
      pltpu.sync_copy(rbs[zc % 2], s_sh.at[pl.ds(base_r + zc * cb, cb)])
    plsc.subcore_barrier()

    if not use_vals:
      one = jnp.ones((16,), jnp.float32)
      for rb in rbs:
        for r in range(cb):
          for q in range(h // 16):
            rb[r, pl.ds(q * 16, 16)] = one

    for bb in range(2):
      off = wid * per_w + bb * cb
      pltpu.async_copy(dst_hbm.at[pl.ds(off, cb)], ibs[bb], sis[bb])
      if use_vals:
        pltpu.async_copy(vals_hbm.at[pl.ds(off, cb)], rbs[bb], srs[bb])

    def chunk_work(jj, bb, may_prefetch):
      off = wid * per_w + jj * cb
      pltpu.make_async_copy(
          dst_hbm.at[pl.ds(off, cb)], ibs[bb], sis[bb]).wait()
      if use_vals:
        pltpu.make_async_copy(
            vals_hbm.at[pl.ds(off, cb)], rbs[bb], srs[bb]).wait()
      pltpu.sync_copy(rbs[bb], s_sh.at[ibs[bb]], add=True)
      if may_prefetch:
        @pl.when(jj + 2 < n_chunks)
        def _():
          pltpu.async_copy(
              dst_hbm.at[pl.ds(off + 2 * cb, cb)], ibs[bb], sis[bb])
          if use_vals:
            pltpu.async_copy(
                vals_hbm.at[pl.ds(off + 2 * cb, cb)], rbs[bb], srs[bb])

    def body(g, carry):
      for bb in range(2):
        chunk_work(g * 2 + bb, bb, True)
      return carry

    lax.fori_loop(0, n_chunks // 2, body, 0)
    if n_chunks % 2:
      chunk_work(n_chunks - 1, (n_chunks - 1) % 2, False)
    plsc.subcore_barrier()

    out_base = cid * n + sid * spacing
    for zc in range(n_zchunks):
      pltpu.sync_copy(s_sh.at[pl.ds(base_r + zc * cb, cb)], rbs[zc % 2])
      pltpu.sync_copy(rbs[zc % 2], sums_hbm.at[pl.ds(out_base + zc * cb, cb)])

  if use_vals:
    out = k(vals, dst)
  else:
    out = k(dst)
  return out.reshape(NC, n, h)


def _elu(t):
  return jnp.where(t > 0, t, jnp.exp(jnp.minimum(t, 0.0)) - 1.0)


def _ln(r, gam, bet):
  mu = jnp.mean(r, axis=-1, keepdims=True)
  d = r - mu
  var = jnp.mean(d * d, axis=-1, keepdims=True)
  return d * lax.rsqrt(var + 1e-5) * gam + bet


def _edge_body(g0, g1, ea, w0c, w1, b0, b1, gam, bet, out):
  eav = ea[...]
  t = (g0[...] + g1[...]
       + jnp.dot(eav, w0c[...], preferred_element_type=jnp.float32)
       + b0[...])
  t = _elu(t)
  t = jnp.dot(t, w1[...], preferred_element_type=jnp.float32) + b1[...]
  out[...] = _ln(eav + t, gam[...], bet[...])


def _edge_mlp(g, ea, w0c, w1, b0, b1, gam, bet):
  e, h = ea.shape
  be = next(bb for bb in (1280, 640, 512, 320) if e % bb == 0)
  grid = e // be
  assert grid * be == e
  wspec = pl.BlockSpec((h, h), lambda i: (0, 0))
  vspec = pl.BlockSpec((1, h), lambda i: (0, 0))
  return pl.pallas_call(
      _edge_body,
      grid=(grid,),
      in_specs=[
          pl.BlockSpec((be, h), lambda i: (i, 0)),
          pl.BlockSpec((be, h), lambda i, g_=grid: (i + g_, 0)),
          pl.BlockSpec((be, h), lambda i: (i, 0)),
          wspec, wspec, vspec, vspec, vspec, vspec,
      ],
      out_specs=pl.BlockSpec((be, h), lambda i: (i, 0)),
      out_shape=jax.ShapeDtypeStruct((e, h), jnp.float32),
      compiler_params=pltpu.CompilerParams(
          dimension_semantics=("arbitrary",)),
  )(g, g, ea, w0c, w1, b0, b1, gam, bet)


def _pre_body(xb, wab, out):
  out[...] = jnp.dot(xb[...], wab[0], preferred_element_type=jnp.float32)


def _pre_products(x, w0ab):
  n, h = x.shape
  bn = 400
  nb = n // bn
  assert nb * bn == n
  return pl.pallas_call(
      _pre_body,
      grid=(2, nb),
      in_specs=[
          pl.BlockSpec((bn, h), lambda m, i: (i, 0)),
          pl.BlockSpec((1, h, h), lambda m, i: (m, 0, 0)),
      ],
      out_specs=pl.BlockSpec((bn, h), lambda m, i, nb_=nb: (m * nb_ + i, 0)),
      out_shape=jax.ShapeDtypeStruct((2 * n, h), jnp.float32),
      compiler_params=pltpu.CompilerParams(
          dimension_semantics=("arbitrary", "arbitrary")),
  )(x, w0ab)


def _node_body(final, xb, sums1, sums2, cnts, wa, wb, w1, b0, b1, gam, bet,
               wo, bo, wnext, out, *extra):
  xv = xb[...]
  s = sums1[0] + sums1[1] + sums2[0] + sums2[1]
  c = jnp.maximum(cnts[0] + cnts[1], 1.0)
  agg = s / c
  t = (jnp.dot(xv, wa[...], preferred_element_type=jnp.float32)
       + jnp.dot(agg, wb[...], preferred_element_type=jnp.float32)
       + b0[...])
  t = _elu(t)
  t = jnp.dot(t, w1[...], preferred_element_type=jnp.float32) + b1[...]
  xn = _ln(xv + t, gam[...], bet[...])
  out[...] = xn
  if final:
    proj = jnp.dot(xn, wo[...], preferred_element_type=jnp.float32) + bo[...]
    extra[0][...] = _elu(proj)
  else:
    wn = wnext[...]
    extra[0][...] = jnp.dot(xn, wn[0], preferred_element_type=jnp.float32)
    extra[1][...] = jnp.dot(xn, wn[1], preferred_element_type=jnp.float32)


def _node_mlp(x, sums1, sums2, cnts, wa, wb, w1, b0, b1, gam, bet, wo, bo,
              wnext, final):
  n, h = x.shape
  bn = 512
  grid = pl.cdiv(n, bn)
  wspec = pl.BlockSpec((h, h), lambda i: (0, 0))
  vspec = pl.BlockSpec((1, h), lambda i: (0, 0))
  out_shape = [jax.ShapeDtypeStruct((n, h), jnp.float32)]
  out_specs = [pl.BlockSpec((bn, h), lambda i: (i, 0))]
  n_extra = 1 if final else 2
  for _ in range(n_extra):
    out_shape.append(jax.ShapeDtypeStruct((n, h), jnp.float32))
    out_specs.append(pl.BlockSpec((bn, h), lambda i: (i, 0)))
  res = pl.pallas_call(
      functools.partial(_node_body, final),
      grid=(grid,),
      in_specs=[
          pl.BlockSpec((bn, h), lambda i: (i, 0)),
          pl.BlockSpec((NC, bn, h), lambda i: (0, i, 0)),
          pl.BlockSpec((NC, bn, h), lambda i: (0, i, 0)),
          pl.BlockSpec((NC, bn, h), lambda i: (0, i, 0)),
          wspec, wspec, wspec, vspec, vspec, vspec, vspec,
          wspec, vspec,
          pl.BlockSpec((2, h, h), lambda i: (0, 0, 0)),
      ],
      out_specs=out_specs,
      out_shape=out_shape,
      compiler_params=pltpu.CompilerParams(
          dimension_semantics=("arbitrary",)),
  )(x, sums1, sums2, cnts, wa, wb, w1, b0, b1, gam, bet, wo, bo, wnext)
  return res


def kernel(x, edge_index, edge_attr, edge_indices, edge_indices_f2c, clusters,
           batches, positions, lengthscales, params):
  n, h = x.shape
  ei = edge_indices[0]
  e = ei.shape[1]
  ek = e // 2
  idx_parts = [jnp.concatenate([ei[0, k * ek:(k + 1) * ek],
                                ei[1, k * ek:(k + 1) * ek] + n])
               for k in range(2)]
  dst_parts = [ei[1, k * ek:(k + 1) * ek] for k in range(2)]

  wout_pad = jnp.zeros((h, h), jnp.float32).at[:, :params['wout'].shape[1]].set(
      params['wout'])
  bout_pad = jnp.zeros((1, h), jnp.float32).at[:, :params['bout'].shape[0]].set(
      params['bout'][None, :])

  def row(v):
    return v.reshape(1, h)

  ea_parts = [edge_attr[k * ek:(k + 1) * ek] for k in range(2)]
  out_proj = None
  n_mp = 2
  w0_stacks = [jnp.stack([params['ew0_%d' % i][:h],
                          params['ew0_%d' % i][h:2 * h]]) for i in range(n_mp)]
  cnts = None
  table = _pre_products(x, w0_stacks[0])
  for i in range(n_mp):
    w0 = params['ew0_%d' % i]
    new_ea = []
    sums_parts = []
    for k in range(2):
      g = _sc_gather(table, idx_parts[k])
      if cnts is None:
        dst_c = jnp.where(g[0, 0] > jnp.float32(3e38), ei[1] ^ 1, ei[1])
        cnts = _sc_scatter_add(None, dst_c, n, h)
      new_ea.append(
          _edge_mlp(g, ea_parts[k], w0[2 * h:],
                    params['ew1_%d' % i], row(params['eb0_%d' % i]),
                    row(params['eb1_%d' % i]), row(params['eg_%d' % i]),
                    row(params['ebt_%d' % i])))
      sums_parts.append(_sc_scatter_add(new_ea[k], dst_parts[k], n, h))
    ea_parts = new_ea
    nw0 = params['nw0_%d' % i]
    final = i == n_mp - 1
    wnext = w0_stacks[0] if final else w0_stacks[i + 1]
    res = _node_mlp(x, sums_parts[0], sums_parts[1], cnts, nw0[:h], nw0[h:],
                    params['nw1_%d' % i], row(params['nb0_%d' % i]),
                    row(params['nb1_%d' % i]), row(params['ng_%d' % i]),
                    row(params['nbt_%d' % i]), wout_pad, bout_pad, wnext,
                    final)
    if final:
      x, out_proj = res
    else:
      x, p_next, q_next = res
      table = jnp.concatenate([p_next, q_next], axis=0)

  return (out_proj[:, :params['wout'].shape[1]], ei)


# speedup vs baseline: 4.1533x; 1.0730x over previous
import functools

import jax
import jax.numpy as jnp
from jax import lax
from jax.experimental import pallas as pl
from jax.experimental.pallas import tpu as pltpu
from jax.experimental.pallas import tpu_sc as plsc

NC = 2
NS = 16
NW = NC * NS


def _sc_gather(table, idx):
  n, h = table.shape
  b = idx.shape[0]
  per_w = b // NW
  cb = next(c for c in (80, 40, 16, 8) if per_w % c == 0)
  n_chunks = per_w // cb
  assert per_w * NW == b and n_chunks * cb == per_w

  mesh = plsc.VectorSubcoreMesh(core_axis_name="c", subcore_axis_name="s")

  assert n_chunks >= 6

  @functools.partial(
      pl.kernel,
      mesh=mesh,
      out_type=jax.ShapeDtypeStruct((b, h), jnp.float32),
      scratch_types=(
          [pltpu.VMEM((cb,), jnp.int32)] * 4
          + [pltpu.VMEM((cb, h), jnp.float32)] * 2
          + [pltpu.SemaphoreType.DMA] * 8
      ),
  )
  def k(table_hbm, idx_hbm, out_hbm, ib0, ib1, ib2, ib3, rb0, rb1,
        si0, si1, si2, si3, sg0, sg1, so0, so1):
    wid = lax.axis_index("s") * NC + lax.axis_index("c")
    base = wid * per_w
    ibs, rbs = (ib0, ib1, ib2, ib3), (rb0, rb1)
    sis, sgs, sos = (si0, si1, si2, si3), (sg0, sg1), (so0, so1)

    def idx_fetch(jj, b4):
      pltpu.async_copy(idx_hbm.at[pl.ds(base + jj * cb, cb)],
                       ibs[b4], sis[b4])

    def gather_start(jj, b2, b4):
      pltpu.make_async_copy(idx_hbm.at[pl.ds(base + jj * cb, cb)],
                            ibs[b4], sis[b4]).wait()
      pltpu.async_copy(table_hbm.at[ibs[b4]], rbs[b2], sgs[b2])

    def out_start(jj, b2):
      pltpu.make_async_copy(table_hbm.at[pl.ds(0, cb)], rbs[b2],
                            sgs[b2]).wait()
      pltpu.async_copy(rbs[b2], out_hbm.at[pl.ds(base + jj * cb, cb)],
                       sos[b2])

    def out_drain(jj, b2):
      pltpu.make_async_copy(rbs[b2], out_hbm.at[pl.ds(base + jj * cb, cb)],
                            sos[b2]).wait()

    for jj in range(4):
      idx_fetch(jj, jj % 4)
    gather_start(0, 0, 0)
    gather_start(1, 1, 1)
    out_start(0, 0)

    def body(g, carry):
      for u in range(4):
        jj = g * 4 + 2 + u
        b2, b4 = u % 2, (2 + u) % 4
        out_drain(jj - 2, b2)
        gather_start(jj, b2, b4)
        out_start(jj - 1, 1 - b2)
        idx_fetch(jj + 2, u % 4)
      return carry

    n_mid = (n_chunks - 4) // 4
    lax.fori_loop(0, n_mid, body, 0)

    for jj in range(2 + n_mid * 4, n_chunks):
      b2, b4 = jj % 2, jj % 4
      out_drain(jj - 2, b2)
      gather_start(jj, b2, b4)
      out_start(jj - 1, 1 - b2)
      if jj + 2 < n_chunks:
        idx_fetch(jj + 2, (jj + 2) % 4)

    last = n_chunks - 1
    out_start(last, last % 2)
    out_drain(last - 1, (last - 1) % 2)
    out_drain(last, last % 2)

  return k(table, idx)


def _sc_scatter_add(vals, dst, n, h):
  e = dst.shape[0]
  per_w = e // NW
  cb = next(c for c in (80, 40, 16, 8) if per_w % c == 0)
  n_chunks = per_w // cb
  zr = 640
  spacing = 624
  assert per_w * NW == e and n_chunks * cb == per_w
  assert spacing % 8 == 0 and spacing <= zr and spacing * (NS - 1) + zr == n
  n_zchunks = zr // cb
  use_vals = vals is not None

  mesh = plsc.VectorSubcoreMesh(core_axis_name="c", subcore_axis_name="s")

  @functools.partial(
      pl.kernel,
      mesh=mesh,
      out_type=jax.ShapeDtypeStruct((NC * n, h), jnp.float32),
      scratch_types=[
          pltpu.VMEM((cb,), jnp.int32),
          pltpu.VMEM((cb,), jnp.int32),
          pltpu.VMEM((cb, h), jnp.float32),
          pltpu.VMEM((cb, h), jnp.float32),
          pltpu.SemaphoreType.DMA,
          pltpu.SemaphoreType.DMA,
          pltpu.SemaphoreType.DMA,
          pltpu.SemaphoreType.DMA,
          pltpu.VMEM_SHARED((n, h), jnp.float32),
      ],
  )
  def k(*refs):
    if use_vals:
      (vals_hbm, dst_hbm, sums_hbm, ib0, ib1, rb0, rb1,
       si0, si1, sr0, sr1, s_sh) = refs
    else:
      (dst_hbm, sums_hbm, ib0, ib1, rb0, rb1,
       si0, si1, sr0, sr1, s_sh) = refs
    cid = lax.axis_index("c")
    sid = lax.axis_index("s")
    wid = sid * NC + cid
    ibs, rbs = (ib0, ib1), (rb0, rb1)
    sis, srs = (si0, si1), (sr0, sr1)

    fill = jnp.zeros((16,), jnp.float32)
    for rb in rbs:
      for r in range(cb):
        for q in range(h // 16):
          rb[r, pl.ds(q * 16, 16)] = fill

    base_r = sid * spacing
    for zc in range(n_zchunks):
      pltpu.sync_copy(rbs[zc % 2], s_sh.at[pl.ds(base_r + zc * cb, cb)])
    plsc.subcore_barrier()

    if not use_vals:
      one = jnp.ones((16,), jnp.float32)
      for rb in rbs:
        for r in range(cb):
          for q in range(h // 16):
            rb[r, pl.ds(q * 16, 16)] = one

    for bb in range(2):
      off = wid * per_w + bb * cb
      pltpu.async_copy(dst_hbm.at[pl.ds(off, cb)], ibs[bb], sis[bb])
      if use_vals:
        pltpu.async_copy(vals_hbm.at[pl.ds(off, cb)], rbs[bb], srs[bb])

    def chunk_work(jj, bb, may_prefetch):
      off = wid * per_w + jj * cb
      pltpu.make_async_copy(
          dst_hbm.at[pl.ds(off, cb)], ibs[bb], sis[bb]).wait()
      if use_vals:
        pltpu.make_async_copy(
            vals_hbm.at[pl.ds(off, cb)], rbs[bb], srs[bb]).wait()
      pltpu.sync_copy(rbs[bb], s_sh.at[ibs[bb]], add=True)
      if may_prefetch:
        @pl.when(jj + 2 < n_chunks)
        def _():
          pltpu.async_copy(
              dst_hbm.at[pl.ds(off + 2 * cb, cb)], ibs[bb], sis[bb])
          if use_vals:
            pltpu.async_copy(
                vals_hbm.at[pl.ds(off + 2 * cb, cb)], rbs[bb], srs[bb])

    def body(g, carry):
      for bb in range(2):
        chunk_work(g * 2 + bb, bb, True)
      return carry

    lax.fori_loop(0, n_chunks // 2, body, 0)
    if n_chunks % 2:
      chunk_work(n_chunks - 1, (n_chunks - 1) % 2, False)
    plsc.subcore_barrier()

    out_base = cid * n + sid * spacing
    for zc in range(n_zchunks):
      pltpu.sync_copy(s_sh.at[pl.ds(base_r + zc * cb, cb)], rbs[zc % 2])
      pltpu.sync_copy(rbs[zc % 2], sums_hbm.at[pl.ds(out_base + zc * cb, cb)])

  if use_vals:
    out = k(vals, dst)
  else:
    out = k(dst)
  return out.reshape(NC, n, h)


def _elu(t):
  return jnp.where(t > 0, t, jnp.exp(jnp.minimum(t, 0.0)) - 1.0)


def _ln(r, gam, bet):
  mu = jnp.mean(r, axis=-1, keepdims=True)
  d = r - mu
  var = jnp.mean(d * d, axis=-1, keepdims=True)
  return d * lax.rsqrt(var + 1e-5) * gam + bet


def _edge_body(g0, g1, ea, w0c, w1, b0, b1, gam, bet, out):
  eav = ea[...]
  t = (g0[...] + g1[...]
       + jnp.dot(eav, w0c[...], preferred_element_type=jnp.float32)
       + b0[...])
  t = _elu(t)
  t = jnp.dot(t, w1[...], preferred_element_type=jnp.float32) + b1[...]
  out[...] = _ln(eav + t, gam[...], bet[...])


def _edge_mlp(g, ea, w0c, w1, b0, b1, gam, bet):
  e, h = ea.shape
  be = next(bb for bb in (3200, 1280, 640, 512, 320) if e % bb == 0)
  grid = e // be
  assert grid * be == e
  wspec = pl.BlockSpec((h, h), lambda i: (0, 0))
  vspec = pl.BlockSpec((1, h), lambda i: (0, 0))
  return pl.pallas_call(
      _edge_body,
      grid=(grid,),
      in_specs=[
          pl.BlockSpec((be, h), lambda i: (i, 0)),
          pl.BlockSpec((be, h), lambda i, g_=grid: (i + g_, 0)),
          pl.BlockSpec((be, h), lambda i: (i, 0)),
          wspec, wspec, vspec, vspec, vspec, vspec,
      ],
      out_specs=pl.BlockSpec((be, h), lambda i: (i, 0)),
      out_shape=jax.ShapeDtypeStruct((e, h), jnp.float32),
      compiler_params=pltpu.CompilerParams(
          dimension_semantics=("arbitrary",)),
  )(g, g, ea, w0c, w1, b0, b1, gam, bet)


def _pre_body(xb, wab, out):
  out[...] = jnp.dot(xb[...], wab[0], preferred_element_type=jnp.float32)


def _pre_products(x, w0ab):
  n, h = x.shape
  bn = 1000 if n % 1000 == 0 else 400
  nb = n // bn
  assert nb * bn == n
  return pl.pallas_call(
      _pre_body,
      grid=(2, nb),
      in_specs=[
          pl.BlockSpec((bn, h), lambda m, i: (i, 0)),
          pl.BlockSpec((1, h, h), lambda m, i: (m, 0, 0)),
      ],
      out_specs=pl.BlockSpec((bn, h), lambda m, i, nb_=nb: (m * nb_ + i, 0)),
      out_shape=jax.ShapeDtypeStruct((2 * n, h), jnp.float32),
      compiler_params=pltpu.CompilerParams(
          dimension_semantics=("arbitrary", "arbitrary")),
  )(x, w0ab)


def _node_body(final, xb, sums1, sums2, cnts, wa, wb, w1, b0, b1, gam, bet,
               wo, bo, wnext, out, *extra):
  xv = xb[...]
  s = sums1[0] + sums1[1] + sums2[0] + sums2[1]
  c = jnp.maximum(cnts[0] + cnts[1], 1.0)
  agg = s / c
  t = (jnp.dot(xv, wa[...], preferred_element_type=jnp.float32)
       + jnp.dot(agg, wb[...], preferred_element_type=jnp.float32)
       + b0[...])
  t = _elu(t)
  t = jnp.dot(t, w1[...], preferred_element_type=jnp.float32) + b1[...]
  xn = _ln(xv + t, gam[...], bet[...])
  out[...] = xn
  if final:
    proj = jnp.dot(xn, wo[...], preferred_element_type=jnp.float32) + bo[...]
    extra[0][...] = _elu(proj)
  else:
    wn = wnext[...]
    extra[0][...] = jnp.dot(xn, wn[0], preferred_element_type=jnp.float32)
    extra[1][...] = jnp.dot(xn, wn[1], preferred_element_type=jnp.float32)


def _node_mlp(x, sums1, sums2, cnts, wa, wb, w1, b0, b1, gam, bet, wo, bo,
              wnext, final):
  n, h = x.shape
  bn = 1000 if n % 1000 == 0 else 512
  grid = pl.cdiv(n, bn)
  wspec = pl.BlockSpec((h, h), lambda i: (0, 0))
  vspec = pl.BlockSpec((1, h), lambda i: (0, 0))
  out_shape = [jax.ShapeDtypeStruct((n, h), jnp.float32)]
  out_specs = [pl.BlockSpec((bn, h), lambda i: (i, 0))]
  n_extra = 1 if final else 2
  for _ in range(n_extra):
    out_shape.append(jax.ShapeDtypeStruct((n, h), jnp.float32))
    out_specs.append(pl.BlockSpec((bn, h), lambda i: (i, 0)))
  res = pl.pallas_call(
      functools.partial(_node_body, final),
      grid=(grid,),
      in_specs=[
          pl.BlockSpec((bn, h), lambda i: (i, 0)),
          pl.BlockSpec((NC, bn, h), lambda i: (0, i, 0)),
          pl.BlockSpec((NC, bn, h), lambda i: (0, i, 0)),
          pl.BlockSpec((NC, bn, h), lambda i: (0, i, 0)),
          wspec, wspec, wspec, vspec, vspec, vspec, vspec,
          wspec, vspec,
          pl.BlockSpec((2, h, h), lambda i: (0, 0, 0)),
      ],
      out_specs=out_specs,
      out_shape=out_shape,
      compiler_params=pltpu.CompilerParams(
          dimension_semantics=("arbitrary",)),
  )(x, sums1, sums2, cnts, wa, wb, w1, b0, b1, gam, bet, wo, bo, wnext)
  return res


def kernel(x, edge_index, edge_attr, edge_indices, edge_indices_f2c, clusters,
           batches, positions, lengthscales, params):
  n, h = x.shape
  ei = edge_indices[0]
  e = ei.shape[1]
  ek = e // 2
  idx_parts = [jnp.concatenate([ei[0, k * ek:(k + 1) * ek],
                                ei[1, k * ek:(k + 1) * ek] + n])
               for k in range(2)]
  dst_parts = [ei[1, k * ek:(k + 1) * ek] for k in range(2)]

  wout_pad = jnp.zeros((h, h), jnp.float32).at[:, :params['wout'].shape[1]].set(
      params['wout'])
  bout_pad = jnp.zeros((1, h), jnp.float32).at[:, :params['bout'].shape[0]].set(
      params['bout'][None, :])

  def row(v):
    return v.reshape(1, h)

  ea_parts = [edge_attr[k * ek:(k + 1) * ek] for k in range(2)]
  out_proj = None
  n_mp = 2
  w0_stacks = [jnp.stack([params['ew0_%d' % i][:h],
                          params['ew0_%d' % i][h:2 * h]]) for i in range(n_mp)]
  cnts = None
  table = _pre_products(x, w0_stacks[0])
  for i in range(n_mp):
    w0 = params['ew0_%d' % i]
    new_ea = []
    sums_parts = []
    for k in range(2):
      g = _sc_gather(table, idx_parts[k])
      if cnts is None:
        dst_c = jnp.where(g[0, 0] > jnp.float32(3e38), ei[1] ^ 1, ei[1])
        cnts = _sc_scatter_add(None, dst_c, n, h)
      new_ea.append(
          _edge_mlp(g, ea_parts[k], w0[2 * h:],
                    params['ew1_%d' % i], row(params['eb0_%d' % i]),
                    row(params['eb1_%d' % i]), row(params['eg_%d' % i]),
                    row(params['ebt_%d' % i])))
      sums_parts.append(_sc_scatter_add(new_ea[k], dst_parts[k], n, h))
    ea_parts = new_ea
    nw0 = params['nw0_%d' % i]
    final = i == n_mp - 1
    wnext = w0_stacks[0] if final else w0_stacks[i + 1]
    res = _node_mlp(x, sums_parts[0], sums_parts[1], cnts, nw0[:h], nw0[h:],
                    params['nw1_%d' % i], row(params['nb0_%d' % i]),
                    row(params['nb1_%d' % i]), row(params['ng_%d' % i]),
                    row(params['nbt_%d' % i]), wout_pad, bout_pad, wnext,
                    final)
    if final:
      x, out_proj = res
    else:
      x, p_next, q_next = res
      table = jnp.concatenate([p_next, q_next], axis=0)

  return (out_proj[:, :params['wout'].shape[1]], ei)
